# Initial kernel scaffold; baseline (speedup 1.0000x reference)
import jax
import jax.numpy as jnp
from jax.experimental import pallas as pl


def kernel(p, x, Wq, bq, Wk, bk, Wv, bv, pe_W1, pe_g1, pe_b1, pe_W2, pe_c2, at_g1, at_b1, at_W1, at_g2, at_b2, at_W2, at_c2):
    raise NotImplementedError("write your pallas kernel here")


# speedup vs baseline: 10.2876x; 10.2876x over previous
import functools

import jax
import jax.numpy as jnp
from jax import lax
from jax.experimental import pallas as pl
from jax.experimental.pallas import tpu as pltpu
from jax.experimental.pallas import tpu_sc as plsc

B, N, C, K, CD = 4, 4096, 128, 16, 3
MN = 256
NB = 512
BIG = 1e30
EPS = 1e-5
BNK = B * N * K


def _qkv_body(x_ref, wqt_ref, wkt_ref, wvt_ref, bq_ref, bk_ref, bv_ref,
              qT_ref, kvT_ref):
    xt = x_ref[0].T
    qT_ref[0] = jnp.dot(xt, wqt_ref[...], preferred_element_type=jnp.float32) + bq_ref[...]
    kvT_ref[0, :, 0:C] = jnp.dot(xt, wkt_ref[...], preferred_element_type=jnp.float32) + bk_ref[...]
    kvT_ref[0, :, C:2 * C] = jnp.dot(xt, wvt_ref[...], preferred_element_type=jnp.float32) + bv_ref[...]


def _qkv_call(x, WqT, WkT, WvT, bq, bk, bv):
    return pl.pallas_call(
        _qkv_body,
        grid=(B, N // NB),
        in_specs=[
            pl.BlockSpec((1, C, NB), lambda b, n: (b, 0, n)),
            pl.BlockSpec((C, C), lambda b, n: (0, 0)),
            pl.BlockSpec((C, C), lambda b, n: (0, 0)),
            pl.BlockSpec((C, C), lambda b, n: (0, 0)),
            pl.BlockSpec((1, C), lambda b, n: (0, 0)),
            pl.BlockSpec((1, C), lambda b, n: (0, 0)),
            pl.BlockSpec((1, C), lambda b, n: (0, 0)),
        ],
        out_specs=[
            pl.BlockSpec((1, NB, C), lambda b, n: (b, n, 0)),
            pl.BlockSpec((1, NB, 2 * C), lambda b, n: (b, n, 0)),
        ],
        out_shape=[
            jax.ShapeDtypeStruct((B, N, C), jnp.float32),
            jax.ShapeDtypeStruct((B, N, 2 * C), jnp.float32),
        ],
        compiler_params=pltpu.CompilerParams(
            dimension_semantics=("arbitrary", "arbitrary")),
    )(x, WqT, WkT, WvT, bq, bk, bv)


def _topk_body(pa_ref, prt_ref, idx_ref, dist_ref):
    b = pl.program_id(0)
    pa = pa_ref[0]
    prt = prt_ref[0]
    sq_all = jnp.sum(pa * pa, axis=1, keepdims=True)
    sq_rows = jnp.sum(prt * prt, axis=0, keepdims=True)
    inner = jnp.dot(pa, prt, preferred_element_type=jnp.float32)
    d = jnp.maximum(sq_all - 2.0 * inner + sq_rows, 0.0)
    zmask = jnp.all(pa == 0.0, axis=1, keepdims=True)
    dist_ref[...] = jnp.where(zmask, BIG, d)
    iota0 = lax.broadcasted_iota(jnp.int32, (N, MN), 0)
    rows16 = lax.broadcasted_iota(jnp.int32, (16, MN), 0)

    def step(kk, acc):
        dd = dist_ref[...]
        m = jnp.min(dd, axis=0, keepdims=True)
        sel = jnp.where(dd == m, iota0, N)
        j = jnp.min(sel, axis=0, keepdims=True)
        dist_ref[...] = jnp.where(iota0 == j, BIG, dd)
        return jnp.where(rows16 == kk, j + b * N, acc)

    idx_ref[0] = lax.fori_loop(0, 16, step, jnp.zeros((16, MN), jnp.int32))


def _topk_call(p_pad8, pT8):
    return pl.pallas_call(
        _topk_body,
        grid=(B, N // MN),
        in_specs=[
            pl.BlockSpec((1, N, 8), lambda b, n: (b, 0, 0)),
            pl.BlockSpec((1, 8, MN), lambda b, n: (b, 0, n)),
        ],
        out_specs=pl.BlockSpec((1, 16, MN), lambda b, n: (b, 0, n)),
        out_shape=jax.ShapeDtypeStruct((B, 16, N), jnp.int32),
        scratch_shapes=[pltpu.VMEM((N, MN), jnp.float32)],
        compiler_params=pltpu.CompilerParams(
            dimension_semantics=("arbitrary", "arbitrary")),
    )(p_pad8, pT8)


_NW = 32
_ROWS_W = BNK // _NW
_CH = 128


def _gather_body(kv_hbm, pp_hbm, idx_hbm, nkv_hbm, np_hbm,
                 idxv, kvbuf, ppbuf, sem1, sem2):
    cid = lax.axis_index("c")
    sid = lax.axis_index("s")
    wid = sid * 2 + cid
    base = wid * _ROWS_W

    def body(i, carry):
        off = base + i * _CH
        pltpu.sync_copy(idx_hbm.at[pl.ds(off, _CH)], idxv)
        cp1 = pltpu.async_copy(kv_hbm.at[idxv], kvbuf, sem1)
        cp2 = pltpu.async_copy(pp_hbm.at[idxv], ppbuf, sem2)
        cp1.wait()
        cp2.wait()
        pltpu.sync_copy(kvbuf, nkv_hbm.at[pl.ds(off, _CH)])
        pltpu.sync_copy(ppbuf, np_hbm.at[pl.ds(off, _CH)])
        return carry

    lax.fori_loop(0, _ROWS_W // _CH, body, 0)


def _gather_call(kv_flat, pp_flat, idx_flat):
    mesh = plsc.VectorSubcoreMesh(core_axis_name="c", subcore_axis_name="s")
    fn = pl.kernel(
        _gather_body,
        out_type=[
            jax.ShapeDtypeStruct((BNK, 2 * C), jnp.float32),
            jax.ShapeDtypeStruct((BNK, 16), jnp.float32),
        ],
        mesh=mesh,
        scratch_types=[
            pltpu.VMEM((_CH,), jnp.int32),
            pltpu.VMEM((_CH, 2 * C), jnp.float32),
            pltpu.VMEM((_CH, 16), jnp.float32),
            pltpu.SemaphoreType.DMA,
            pltpu.SemaphoreType.DMA,
        ],
        compiler_params=pltpu.CompilerParams(use_tc_tiling_on_sc=False),
    )
    return fn(kv_flat, pp_flat, idx_flat)


def _zstats_body(p16_ref, npp_ref, w1t_ref, sumz_ref, sumsq_ref):
    @pl.when(jnp.logical_and(pl.program_id(0) == 0, pl.program_id(1) == 0))
    def _():
        sumz_ref[...] = jnp.zeros_like(sumz_ref)
        sumsq_ref[...] = jnp.zeros_like(sumsq_ref)

    ssum = jnp.zeros((1, C), jnp.float32)
    ssq = jnp.zeros((1, C), jnp.float32)
    prow = p16_ref[0]
    for k in range(K):
        r_k = prow - npp_ref[0, k]
        z = jnp.dot(r_k, w1t_ref[...], preferred_element_type=jnp.float32)
        ssum = ssum + jnp.sum(z, axis=0, keepdims=True)
        ssq = ssq + jnp.sum(z * z, axis=0, keepdims=True)
    sumz_ref[...] += ssum
    sumsq_ref[...] += ssq


def _zstats_call(p16, npp, W1T_pad):
    return pl.pallas_call(
        _zstats_body,
        grid=(B, N // NB),
        in_specs=[
            pl.BlockSpec((1, NB, 16), lambda b, n: (b, n, 0)),
            pl.BlockSpec((1, K, NB, 16), lambda b, n: (b, 0, n, 0)),
            pl.BlockSpec((16, C), lambda b, n: (0, 0)),
        ],
        out_specs=[
            pl.BlockSpec((1, C), lambda b, n: (0, 0)),
            pl.BlockSpec((1, C), lambda b, n: (0, 0)),
        ],
        out_shape=[
            jax.ShapeDtypeStruct((1, C), jnp.float32),
            jax.ShapeDtypeStruct((1, C), jnp.float32),
        ],
        compiler_params=pltpu.CompilerParams(
            dimension_semantics=("arbitrary", "arbitrary")),
    )(p16, npp, W1T_pad)


def _p1_body(p16_ref, npp_ref, nkv_ref, qT_ref, w1t_ref, s1_ref, t1_ref,
             w2t_ref, c2_ref, apre_ref, nr_ref, sum2_ref, sumsq2_ref):
    @pl.when(jnp.logical_and(pl.program_id(0) == 0, pl.program_id(1) == 0))
    def _():
        sum2_ref[...] = jnp.zeros_like(sum2_ref)
        sumsq2_ref[...] = jnp.zeros_like(sumsq2_ref)

    prow = p16_ref[0]
    qrow = qT_ref[0]
    s1 = s1_ref[...]
    t1 = t1_ref[...]
    c2 = c2_ref[...]
    ssum = jnp.zeros((1, C), jnp.float32)
    ssq = jnp.zeros((1, C), jnp.float32)
    for k in range(K):
        r_k = prow - npp_ref[0, k]
        z = jnp.dot(r_k, w1t_ref[...], preferred_element_type=jnp.float32)
        h1 = jnp.maximum(z * s1 + t1, 0.0)
        nr_k = jnp.dot(h1, w2t_ref[...], preferred_element_type=jnp.float32) + c2
        apre_k = qrow - nkv_ref[0, k] + nr_k
        nr_ref[0, k] = nr_k
        apre_ref[0, k] = apre_k
        ssum = ssum + jnp.sum(apre_k, axis=0, keepdims=True)
        ssq = ssq + jnp.sum(apre_k * apre_k, axis=0, keepdims=True)
    sum2_ref[...] += ssum
    sumsq2_ref[...] += ssq


def _p1_call(p16, npp, nkv, qT, W1T_pad, s1, t1, W2T, c2):
    return pl.pallas_call(
        _p1_body,
        grid=(B, N // NB),
        in_specs=[
            pl.BlockSpec((1, NB, 16), lambda b, n: (b, n, 0)),
            pl.BlockSpec((1, K, NB, 16), lambda b, n: (b, 0, n, 0)),
            pl.BlockSpec((1, K, NB, C), lambda b, n: (b, 0, n, 0)),
            pl.BlockSpec((1, NB, C), lambda b, n: (b, n, 0)),
            pl.BlockSpec((16, C), lambda b, n: (0, 0)),
            pl.BlockSpec((1, C), lambda b, n: (0, 0)),
            pl.BlockSpec((1, C), lambda b, n: (0, 0)),
            pl.BlockSpec((C, C), lambda b, n: (0, 0)),
            pl.BlockSpec((1, C), lambda b, n: (0, 0)),
        ],
        out_specs=[
            pl.BlockSpec((1, K, NB, C), lambda b, n: (b, 0, n, 0)),
            pl.BlockSpec((1, K, NB, C), lambda b, n: (b, 0, n, 0)),
            pl.BlockSpec((1, C), lambda b, n: (0, 0)),
            pl.BlockSpec((1, C), lambda b, n: (0, 0)),
        ],
        out_shape=[
            jax.ShapeDtypeStruct((B, K, N, C), jnp.float32),
            jax.ShapeDtypeStruct((B, K, N, C), jnp.float32),
            jax.ShapeDtypeStruct((1, C), jnp.float32),
            jax.ShapeDtypeStruct((1, C), jnp.float32),
        ],
        compiler_params=pltpu.CompilerParams(
            dimension_semantics=("arbitrary", "arbitrary")),
    )(p16, npp, nkv, qT, W1T_pad, s1, t1, W2T, c2)


def _p2_body(apre_ref, s2_ref, t2_ref, w1t_ref, a1_ref, sumh_ref, g_ref):
    @pl.when(jnp.logical_and(pl.program_id(0) == 0, pl.program_id(1) == 0))
    def _():
        sumh_ref[...] = jnp.zeros_like(sumh_ref)
        g_ref[...] = jnp.zeros_like(g_ref)

    s2 = s2_ref[...]
    t2 = t2_ref[...]
    ssum = jnp.zeros((1, C), jnp.float32)
    gacc = jnp.zeros((C, C), jnp.float32)
    for k in range(K):
        h = jnp.maximum(apre_ref[0, k] * s2 + t2, 0.0)
        a1_ref[0, k] = jnp.dot(h, w1t_ref[...], preferred_element_type=jnp.float32)
        ssum = ssum + jnp.sum(h, axis=0, keepdims=True)
        gacc = gacc + jnp.dot(h.T, h, preferred_element_type=jnp.float32)
    sumh_ref[...] += ssum
    g_ref[...] += gacc


def _p2_call(apre, s2, t2, atW1T):
    return pl.pallas_call(
        _p2_body,
        grid=(B, N // NB),
        in_specs=[
            pl.BlockSpec((1, K, NB, C), lambda b, n: (b, 0, n, 0)),
            pl.BlockSpec((1, C), lambda b, n: (0, 0)),
            pl.BlockSpec((1, C), lambda b, n: (0, 0)),
            pl.BlockSpec((C, C), lambda b, n: (0, 0)),
        ],
        out_specs=[
            pl.BlockSpec((1, K, NB, C), lambda b, n: (b, 0, n, 0)),
            pl.BlockSpec((1, C), lambda b, n: (0, 0)),
            pl.BlockSpec((C, C), lambda b, n: (0, 0)),
        ],
        out_shape=[
            jax.ShapeDtypeStruct((B, K, N, C), jnp.float32),
            jax.ShapeDtypeStruct((1, C), jnp.float32),
            jax.ShapeDtypeStruct((C, C), jnp.float32),
        ],
        compiler_params=pltpu.CompilerParams(
            dimension_semantics=("arbitrary", "arbitrary")),
    )(apre, s2, t2, atW1T)


def _p3_body(a1_ref, nr_ref, nkv_ref, s3_ref, t3_ref, w2t_ref, c2_ref, y_ref):
    s3 = s3_ref[...]
    t3 = t3_ref[...]
    c2 = c2_ref[...]
    a2 = []
    for k in range(K):
        h = jnp.maximum(a1_ref[0, k] * s3 + t3, 0.0)
        a2.append(jnp.dot(h, w2t_ref[...], preferred_element_type=jnp.float32) + c2)
    mx = a2[0]
    for k in range(1, K):
        mx = jnp.maximum(mx, a2[k])
    esum = jnp.zeros_like(mx)
    ynum = jnp.zeros_like(mx)
    for k in range(K):
        e = jnp.exp(a2[k] - mx)
        esum = esum + e
        ynum = ynum + e * (nkv_ref[0, k] + nr_ref[0, k])
    y = ynum / esum
    y_ref[0] = y.T


def _p3_call(a1, nr, nkv, s3, t3, atW2T, atc2):
    return pl.pallas_call(
        _p3_body,
        grid=(B, N // NB),
        in_specs=[
            pl.BlockSpec((1, K, NB, C), lambda b, n: (b, 0, n, 0)),
            pl.BlockSpec((1, K, NB, C), lambda b, n: (b, 0, n, 0)),
            pl.BlockSpec((1, K, NB, C), lambda b, n: (b, 0, n, 1)),
            pl.BlockSpec((1, C), lambda b, n: (0, 0)),
            pl.BlockSpec((1, C), lambda b, n: (0, 0)),
            pl.BlockSpec((C, C), lambda b, n: (0, 0)),
            pl.BlockSpec((1, C), lambda b, n: (0, 0)),
        ],
        out_specs=pl.BlockSpec((1, C, NB), lambda b, n: (b, 0, n)),
        out_shape=jax.ShapeDtypeStruct((B, C, N), jnp.float32),
        compiler_params=pltpu.CompilerParams(
            dimension_semantics=("arbitrary", "arbitrary")),
    )(a1, nr, nkv, s3, t3, atW2T, atc2)


def _bn_coef(g, b, mean, var):
    s = g / jnp.sqrt(var + EPS)
    return (s[None, :], (b - mean * s)[None, :])


def kernel(p, x, Wq, bq, Wk, bk, Wv, bv, pe_W1, pe_g1, pe_b1, pe_W2, pe_c2,
           at_g1, at_b1, at_W1, at_g2, at_b2, at_W2, at_c2):
    f32 = jnp.float32
    p = p.astype(f32)
    p_pad8 = jnp.pad(p, ((0, 0), (0, 0), (0, 8 - CD)))
    pT8 = jnp.transpose(p_pad8, (0, 2, 1))
    p16 = jnp.pad(p, ((0, 0), (0, 0), (0, 16 - CD)))
    W1T_pad = jnp.pad(pe_W1.T, ((0, 16 - CD), (0, 0)))

    qT, kvT = _qkv_call(x, Wq.T, Wk.T, Wv.T, bq[None, :], bk[None, :], bv[None, :])
    idxT = _topk_call(p_pad8, pT8)

    nkv_flat, npp_flat = _gather_call(
        kvT.reshape(B * N, 2 * C), p16.reshape(B * N, 16), idxT.reshape(-1))
    nkv = nkv_flat.reshape(B, K, N, 2 * C)
    npp = npp_flat.reshape(B, K, N, 16)

    sumz, sumzsq = _zstats_call(p16, npp, W1T_pad)
    mean1 = sumz[0] / BNK
    var1 = sumzsq[0] / BNK - mean1 * mean1
    s1, t1 = _bn_coef(pe_g1, pe_b1, mean1, var1)

    apre, nr, sum2, sumsq2 = _p1_call(
        p16, npp, nkv, qT, W1T_pad, s1, t1, pe_W2.T, pe_c2[None, :])
    mean2 = sum2[0] / BNK
    var2 = sumsq2[0] / BNK - mean2 * mean2
    s2, t2 = _bn_coef(at_g1, at_b1, mean2, var2)

    a1, sumh, G = _p2_call(apre, s2, t2, at_W1.T)
    mu_h = sumh[0] / BNK
    mean3 = at_W1 @ mu_h
    e2 = jnp.sum((at_W1 @ G) * at_W1, axis=1) / BNK
    var3 = e2 - mean3 * mean3
    s3, t3 = _bn_coef(at_g2, at_b2, mean3, var3)

    y = _p3_call(a1, nr, nkv, s3, t3, at_W2.T, at_c2[None, :])
    return (p, y)


# speedup vs baseline: 10.5227x; 1.0229x over previous
import functools

import jax
import jax.numpy as jnp
from jax import lax
from jax.experimental import pallas as pl
from jax.experimental.pallas import tpu as pltpu
from jax.experimental.pallas import tpu_sc as plsc

B, N, C, K, CD = 4, 4096, 128, 16, 3
MN = 512
NB = 512
BIG = 1e30
EPS = 1e-5
BNK = B * N * K


def _qkv_body(x_ref, wqt_ref, wkt_ref, wvt_ref, bq_ref, bk_ref, bv_ref,
              qT_ref, kvT_ref):
    xt = x_ref[0].T
    qT_ref[0] = jnp.dot(xt, wqt_ref[...], preferred_element_type=jnp.float32) + bq_ref[...]
    kvT_ref[0, :, 0:C] = jnp.dot(xt, wkt_ref[...], preferred_element_type=jnp.float32) + bk_ref[...]
    kvT_ref[0, :, C:2 * C] = jnp.dot(xt, wvt_ref[...], preferred_element_type=jnp.float32) + bv_ref[...]


def _qkv_call(x, WqT, WkT, WvT, bq, bk, bv):
    return pl.pallas_call(
        _qkv_body,
        grid=(B, N // NB),
        in_specs=[
            pl.BlockSpec((1, C, NB), lambda b, n: (b, 0, n)),
            pl.BlockSpec((C, C), lambda b, n: (0, 0)),
            pl.BlockSpec((C, C), lambda b, n: (0, 0)),
            pl.BlockSpec((C, C), lambda b, n: (0, 0)),
            pl.BlockSpec((1, C), lambda b, n: (0, 0)),
            pl.BlockSpec((1, C), lambda b, n: (0, 0)),
            pl.BlockSpec((1, C), lambda b, n: (0, 0)),
        ],
        out_specs=[
            pl.BlockSpec((1, NB, C), lambda b, n: (b, n, 0)),
            pl.BlockSpec((1, NB, 2 * C), lambda b, n: (b, n, 0)),
        ],
        out_shape=[
            jax.ShapeDtypeStruct((B, N, C), jnp.float32),
            jax.ShapeDtypeStruct((B, N, 2 * C), jnp.float32),
        ],
        compiler_params=pltpu.CompilerParams(
            dimension_semantics=("arbitrary", "arbitrary")),
    )(x, WqT, WkT, WvT, bq, bk, bv)


def _topk_body(pa_ref, prt_ref, idx_ref, dist_ref):
    b = pl.program_id(0)
    pa = pa_ref[0]
    prt = prt_ref[0]
    sq_all = jnp.sum(pa * pa, axis=1, keepdims=True)
    sq_rows = jnp.sum(prt * prt, axis=0, keepdims=True)
    inner = jnp.dot(pa, prt, preferred_element_type=jnp.float32)
    d = jnp.maximum(sq_all - 2.0 * inner + sq_rows, 0.0)
    zmask = jnp.all(pa == 0.0, axis=1, keepdims=True)
    dist_ref[...] = jnp.where(zmask, BIG, d)
    iota0 = lax.broadcasted_iota(jnp.int32, (N, MN), 0)
    rows16 = lax.broadcasted_iota(jnp.int32, (16, MN), 0)

    def step(kk, acc):
        dd = dist_ref[...]
        m = jnp.min(dd, axis=0, keepdims=True)
        sel = jnp.where(dd == m, iota0, N)
        j = jnp.min(sel, axis=0, keepdims=True)
        dist_ref[...] = jnp.where(iota0 == j, BIG, dd)
        return jnp.where(rows16 == kk, j + b * N, acc)

    idx_ref[0] = lax.fori_loop(0, 16, step, jnp.zeros((16, MN), jnp.int32))


def _topk_call(p_pad8, pT8):
    return pl.pallas_call(
        _topk_body,
        grid=(B, N // MN),
        in_specs=[
            pl.BlockSpec((1, N, 8), lambda b, n: (b, 0, 0)),
            pl.BlockSpec((1, 8, MN), lambda b, n: (b, 0, n)),
        ],
        out_specs=pl.BlockSpec((1, 16, MN), lambda b, n: (b, 0, n)),
        out_shape=jax.ShapeDtypeStruct((B, 16, N), jnp.int32),
        scratch_shapes=[pltpu.VMEM((N, MN), jnp.float32)],
        compiler_params=pltpu.CompilerParams(
            dimension_semantics=("arbitrary", "arbitrary")),
    )(p_pad8, pT8)


_NW = 32
_ROWS_W = BNK // _NW
_CH = 128


def _gather_body(kv_hbm, pp_hbm, idx_hbm, nkv_hbm, np_hbm,
                 idxv, kvbuf, ppbuf, sem1, sem2):
    cid = lax.axis_index("c")
    sid = lax.axis_index("s")
    wid = sid * 2 + cid
    base = wid * _ROWS_W

    def body(i, carry):
        off = base + i * _CH
        pltpu.sync_copy(idx_hbm.at[pl.ds(off, _CH)], idxv)
        cp1 = pltpu.async_copy(kv_hbm.at[idxv], kvbuf, sem1)
        cp2 = pltpu.async_copy(pp_hbm.at[idxv], ppbuf, sem2)
        cp1.wait()
        cp2.wait()
        pltpu.sync_copy(kvbuf, nkv_hbm.at[pl.ds(off, _CH)])
        pltpu.sync_copy(ppbuf, np_hbm.at[pl.ds(off, _CH)])
        return carry

    lax.fori_loop(0, _ROWS_W // _CH, body, 0)


def _gather_call(kv_flat, pp_flat, idx_flat):
    mesh = plsc.VectorSubcoreMesh(core_axis_name="c", subcore_axis_name="s")
    fn = pl.kernel(
        _gather_body,
        out_type=[
            jax.ShapeDtypeStruct((BNK, 2 * C), jnp.float32),
            jax.ShapeDtypeStruct((BNK, 16), jnp.float32),
        ],
        mesh=mesh,
        scratch_types=[
            pltpu.VMEM((_CH,), jnp.int32),
            pltpu.VMEM((_CH, 2 * C), jnp.float32),
            pltpu.VMEM((_CH, 16), jnp.float32),
            pltpu.SemaphoreType.DMA,
            pltpu.SemaphoreType.DMA,
        ],
        compiler_params=pltpu.CompilerParams(use_tc_tiling_on_sc=False),
    )
    return fn(kv_flat, pp_flat, idx_flat)


def _zstats_body(p16_ref, npp_ref, w1t_ref, sumz_ref, sumsq_ref):
    @pl.when(jnp.logical_and(pl.program_id(0) == 0, pl.program_id(1) == 0))
    def _():
        sumz_ref[...] = jnp.zeros_like(sumz_ref)
        sumsq_ref[...] = jnp.zeros_like(sumsq_ref)

    ssum = jnp.zeros((1, C), jnp.float32)
    ssq = jnp.zeros((1, C), jnp.float32)
    prow = p16_ref[0]
    for k in range(K):
        r_k = prow - npp_ref[0, k]
        z = jnp.dot(r_k, w1t_ref[...], preferred_element_type=jnp.float32)
        ssum = ssum + jnp.sum(z, axis=0, keepdims=True)
        ssq = ssq + jnp.sum(z * z, axis=0, keepdims=True)
    sumz_ref[...] += ssum
    sumsq_ref[...] += ssq


def _zstats_call(p16, npp, W1T_pad):
    return pl.pallas_call(
        _zstats_body,
        grid=(B, N // NB),
        in_specs=[
            pl.BlockSpec((1, NB, 16), lambda b, n: (b, n, 0)),
            pl.BlockSpec((1, K, NB, 16), lambda b, n: (b, 0, n, 0)),
            pl.BlockSpec((16, C), lambda b, n: (0, 0)),
        ],
        out_specs=[
            pl.BlockSpec((1, C), lambda b, n: (0, 0)),
            pl.BlockSpec((1, C), lambda b, n: (0, 0)),
        ],
        out_shape=[
            jax.ShapeDtypeStruct((1, C), jnp.float32),
            jax.ShapeDtypeStruct((1, C), jnp.float32),
        ],
        compiler_params=pltpu.CompilerParams(
            dimension_semantics=("arbitrary", "arbitrary")),
    )(p16, npp, W1T_pad)


def _p1_body(p16_ref, npp_ref, nkv_ref, qT_ref, w1t_ref, s1_ref, t1_ref,
             w2t_ref, c2_ref, apre_ref, nr_ref, sum2_ref, sumsq2_ref):
    @pl.when(jnp.logical_and(pl.program_id(0) == 0, pl.program_id(1) == 0))
    def _():
        sum2_ref[...] = jnp.zeros_like(sum2_ref)
        sumsq2_ref[...] = jnp.zeros_like(sumsq2_ref)

    prow = p16_ref[0]
    qrow = qT_ref[0]
    s1 = s1_ref[...]
    t1 = t1_ref[...]
    c2 = c2_ref[...]
    ssum = jnp.zeros((1, C), jnp.float32)
    ssq = jnp.zeros((1, C), jnp.float32)
    for k in range(K):
        r_k = prow - npp_ref[0, k]
        z = jnp.dot(r_k, w1t_ref[...], preferred_element_type=jnp.float32)
        h1 = jnp.maximum(z * s1 + t1, 0.0)
        nr_k = jnp.dot(h1, w2t_ref[...], preferred_element_type=jnp.float32) + c2
        apre_k = qrow - nkv_ref[0, k] + nr_k
        nr_ref[0, k] = nr_k
        apre_ref[0, k] = apre_k
        ssum = ssum + jnp.sum(apre_k, axis=0, keepdims=True)
        ssq = ssq + jnp.sum(apre_k * apre_k, axis=0, keepdims=True)
    sum2_ref[...] += ssum
    sumsq2_ref[...] += ssq


def _p1_call(p16, npp, nkv, qT, W1T_pad, s1, t1, W2T, c2):
    return pl.pallas_call(
        _p1_body,
        grid=(B, N // NB),
        in_specs=[
            pl.BlockSpec((1, NB, 16), lambda b, n: (b, n, 0)),
            pl.BlockSpec((1, K, NB, 16), lambda b, n: (b, 0, n, 0)),
            pl.BlockSpec((1, K, NB, C), lambda b, n: (b, 0, n, 0)),
            pl.BlockSpec((1, NB, C), lambda b, n: (b, n, 0)),
            pl.BlockSpec((16, C), lambda b, n: (0, 0)),
            pl.BlockSpec((1, C), lambda b, n: (0, 0)),
            pl.BlockSpec((1, C), lambda b, n: (0, 0)),
            pl.BlockSpec((C, C), lambda b, n: (0, 0)),
            pl.BlockSpec((1, C), lambda b, n: (0, 0)),
        ],
        out_specs=[
            pl.BlockSpec((1, K, NB, C), lambda b, n: (b, 0, n, 0)),
            pl.BlockSpec((1, K, NB, C), lambda b, n: (b, 0, n, 0)),
            pl.BlockSpec((1, C), lambda b, n: (0, 0)),
            pl.BlockSpec((1, C), lambda b, n: (0, 0)),
        ],
        out_shape=[
            jax.ShapeDtypeStruct((B, K, N, C), jnp.float32),
            jax.ShapeDtypeStruct((B, K, N, C), jnp.float32),
            jax.ShapeDtypeStruct((1, C), jnp.float32),
            jax.ShapeDtypeStruct((1, C), jnp.float32),
        ],
        compiler_params=pltpu.CompilerParams(
            dimension_semantics=("arbitrary", "arbitrary")),
    )(p16, npp, nkv, qT, W1T_pad, s1, t1, W2T, c2)


def _p2_body(apre_ref, s2_ref, t2_ref, w1t_ref, a1_ref, sumh_ref, g_ref):
    @pl.when(jnp.logical_and(pl.program_id(0) == 0, pl.program_id(1) == 0))
    def _():
        sumh_ref[...] = jnp.zeros_like(sumh_ref)
        g_ref[...] = jnp.zeros_like(g_ref)

    s2 = s2_ref[...]
    t2 = t2_ref[...]
    ssum = jnp.zeros((1, C), jnp.float32)
    gacc = jnp.zeros((C, C), jnp.float32)
    for k in range(K):
        h = jnp.maximum(apre_ref[0, k] * s2 + t2, 0.0)
        a1_ref[0, k] = jnp.dot(h, w1t_ref[...], preferred_element_type=jnp.float32)
        ssum = ssum + jnp.sum(h, axis=0, keepdims=True)
        gacc = gacc + jnp.dot(h.T, h, preferred_element_type=jnp.float32)
    sumh_ref[...] += ssum
    g_ref[...] += gacc


def _p2_call(apre, s2, t2, atW1T):
    return pl.pallas_call(
        _p2_body,
        grid=(B, N // NB),
        in_specs=[
            pl.BlockSpec((1, K, NB, C), lambda b, n: (b, 0, n, 0)),
            pl.BlockSpec((1, C), lambda b, n: (0, 0)),
            pl.BlockSpec((1, C), lambda b, n: (0, 0)),
            pl.BlockSpec((C, C), lambda b, n: (0, 0)),
        ],
        out_specs=[
            pl.BlockSpec((1, K, NB, C), lambda b, n: (b, 0, n, 0)),
            pl.BlockSpec((1, C), lambda b, n: (0, 0)),
            pl.BlockSpec((C, C), lambda b, n: (0, 0)),
        ],
        out_shape=[
            jax.ShapeDtypeStruct((B, K, N, C), jnp.float32),
            jax.ShapeDtypeStruct((1, C), jnp.float32),
            jax.ShapeDtypeStruct((C, C), jnp.float32),
        ],
        compiler_params=pltpu.CompilerParams(
            dimension_semantics=("arbitrary", "arbitrary")),
    )(apre, s2, t2, atW1T)


def _p3_body(a1_ref, nr_ref, nkv_ref, s3_ref, t3_ref, w2t_ref, c2_ref, y_ref):
    s3 = s3_ref[...]
    t3 = t3_ref[...]
    c2 = c2_ref[...]
    a2 = []
    for k in range(K):
        h = jnp.maximum(a1_ref[0, k] * s3 + t3, 0.0)
        a2.append(jnp.dot(h, w2t_ref[...], preferred_element_type=jnp.float32) + c2)
    mx = a2[0]
    for k in range(1, K):
        mx = jnp.maximum(mx, a2[k])
    esum = jnp.zeros_like(mx)
    ynum = jnp.zeros_like(mx)
    for k in range(K):
        e = jnp.exp(a2[k] - mx)
        esum = esum + e
        ynum = ynum + e * (nkv_ref[0, k] + nr_ref[0, k])
    y = ynum / esum
    y_ref[0] = y.T


def _p3_call(a1, nr, nkv, s3, t3, atW2T, atc2):
    return pl.pallas_call(
        _p3_body,
        grid=(B, N // NB),
        in_specs=[
            pl.BlockSpec((1, K, NB, C), lambda b, n: (b, 0, n, 0)),
            pl.BlockSpec((1, K, NB, C), lambda b, n: (b, 0, n, 0)),
            pl.BlockSpec((1, K, NB, C), lambda b, n: (b, 0, n, 1)),
            pl.BlockSpec((1, C), lambda b, n: (0, 0)),
            pl.BlockSpec((1, C), lambda b, n: (0, 0)),
            pl.BlockSpec((C, C), lambda b, n: (0, 0)),
            pl.BlockSpec((1, C), lambda b, n: (0, 0)),
        ],
        out_specs=pl.BlockSpec((1, C, NB), lambda b, n: (b, 0, n)),
        out_shape=jax.ShapeDtypeStruct((B, C, N), jnp.float32),
        compiler_params=pltpu.CompilerParams(
            dimension_semantics=("arbitrary", "arbitrary")),
    )(a1, nr, nkv, s3, t3, atW2T, atc2)


def _bn_coef(g, b, mean, var):
    s = g / jnp.sqrt(var + EPS)
    return (s[None, :], (b - mean * s)[None, :])


def kernel(p, x, Wq, bq, Wk, bk, Wv, bv, pe_W1, pe_g1, pe_b1, pe_W2, pe_c2,
           at_g1, at_b1, at_W1, at_g2, at_b2, at_W2, at_c2):
    f32 = jnp.float32
    p = p.astype(f32)
    p_pad8 = jnp.pad(p, ((0, 0), (0, 0), (0, 8 - CD)))
    pT8 = jnp.transpose(p_pad8, (0, 2, 1))
    p16 = jnp.pad(p, ((0, 0), (0, 0), (0, 16 - CD)))
    W1T_pad = jnp.pad(pe_W1.T, ((0, 16 - CD), (0, 0)))

    qT, kvT = _qkv_call(x, Wq.T, Wk.T, Wv.T, bq[None, :], bk[None, :], bv[None, :])
    idxT = _topk_call(p_pad8, pT8)

    nkv_flat, npp_flat = _gather_call(
        kvT.reshape(B * N, 2 * C), p16.reshape(B * N, 16), idxT.reshape(-1))
    nkv = nkv_flat.reshape(B, K, N, 2 * C)
    npp = npp_flat.reshape(B, K, N, 16)

    sumz, sumzsq = _zstats_call(p16, npp, W1T_pad)
    mean1 = sumz[0] / BNK
    var1 = sumzsq[0] / BNK - mean1 * mean1
    s1, t1 = _bn_coef(pe_g1, pe_b1, mean1, var1)

    apre, nr, sum2, sumsq2 = _p1_call(
        p16, npp, nkv, qT, W1T_pad, s1, t1, pe_W2.T, pe_c2[None, :])
    mean2 = sum2[0] / BNK
    var2 = sumsq2[0] / BNK - mean2 * mean2
    s2, t2 = _bn_coef(at_g1, at_b1, mean2, var2)

    a1, sumh, G = _p2_call(apre, s2, t2, at_W1.T)
    mu_h = sumh[0] / BNK
    mean3 = at_W1 @ mu_h
    e2 = jnp.sum((at_W1 @ G) * at_W1, axis=1) / BNK
    var3 = e2 - mean3 * mean3
    s3, t3 = _bn_coef(at_g2, at_b2, mean3, var3)

    y = _p3_call(a1, nr, nkv, s3, t3, at_W2.T, at_c2[None, :])
    return (p, y)


# speedup vs baseline: 11.0745x; 1.0524x over previous
import functools

import jax
import jax.numpy as jnp
from jax import lax
from jax.experimental import pallas as pl
from jax.experimental.pallas import tpu as pltpu
from jax.experimental.pallas import tpu_sc as plsc

B, N, C, K, CD = 4, 4096, 128, 16, 3
MN = 512
NB = 512
BIG = 1e30
EPS = 1e-5
BNK = B * N * K


def _qkv_body(x_ref, wqt_ref, wkt_ref, wvt_ref, bq_ref, bk_ref, bv_ref,
              qT_ref, kvT_ref):
    xt = x_ref[0].T
    qT_ref[0] = jnp.dot(xt, wqt_ref[...], preferred_element_type=jnp.float32) + bq_ref[...]
    kvT_ref[0, :, 0:C] = jnp.dot(xt, wkt_ref[...], preferred_element_type=jnp.float32) + bk_ref[...]
    kvT_ref[0, :, C:2 * C] = jnp.dot(xt, wvt_ref[...], preferred_element_type=jnp.float32) + bv_ref[...]


def _qkv_call(x, WqT, WkT, WvT, bq, bk, bv):
    return pl.pallas_call(
        _qkv_body,
        grid=(B, N // NB),
        in_specs=[
            pl.BlockSpec((1, C, NB), lambda b, n: (b, 0, n)),
            pl.BlockSpec((C, C), lambda b, n: (0, 0)),
            pl.BlockSpec((C, C), lambda b, n: (0, 0)),
            pl.BlockSpec((C, C), lambda b, n: (0, 0)),
            pl.BlockSpec((1, C), lambda b, n: (0, 0)),
            pl.BlockSpec((1, C), lambda b, n: (0, 0)),
            pl.BlockSpec((1, C), lambda b, n: (0, 0)),
        ],
        out_specs=[
            pl.BlockSpec((1, NB, C), lambda b, n: (b, n, 0)),
            pl.BlockSpec((1, NB, 2 * C), lambda b, n: (b, n, 0)),
        ],
        out_shape=[
            jax.ShapeDtypeStruct((B, N, C), jnp.float32),
            jax.ShapeDtypeStruct((B, N, 2 * C), jnp.float32),
        ],
        compiler_params=pltpu.CompilerParams(
            dimension_semantics=("arbitrary", "arbitrary")),
    )(x, WqT, WkT, WvT, bq, bk, bv)


def _topk_body(boff, pa_ref, prt_ref, idx_ref, dist_ref):
    b = pl.program_id(0) + boff
    pa = pa_ref[0]
    prt = prt_ref[0]
    sq_all = jnp.sum(pa * pa, axis=1, keepdims=True)
    sq_rows = jnp.sum(prt * prt, axis=0, keepdims=True)
    inner = jnp.dot(pa, prt, preferred_element_type=jnp.float32)
    d = jnp.maximum(sq_all - 2.0 * inner + sq_rows, 0.0)
    zmask = jnp.all(pa == 0.0, axis=1, keepdims=True)
    dist_ref[...] = jnp.where(zmask, BIG, d)
    iota0 = lax.broadcasted_iota(jnp.int32, (N, MN), 0)
    rows16 = lax.broadcasted_iota(jnp.int32, (16, MN), 0)

    def step(kk, acc):
        dd = dist_ref[...]
        m = jnp.min(dd, axis=0, keepdims=True)
        sel = jnp.where(dd == m, iota0, N)
        j = jnp.min(sel, axis=0, keepdims=True)
        dist_ref[...] = jnp.where(iota0 == j, BIG, dd)
        return jnp.where(rows16 == kk, j + b * N, acc)

    idx_ref[0] = lax.fori_loop(0, 16, step, jnp.zeros((16, MN), jnp.int32))


def _topk_call(p_pad8, pT8, boff):
    nb = p_pad8.shape[0]
    return pl.pallas_call(
        functools.partial(_topk_body, boff),
        grid=(nb, N // MN),
        in_specs=[
            pl.BlockSpec((1, N, 8), lambda b, n: (b, 0, 0)),
            pl.BlockSpec((1, 8, MN), lambda b, n: (b, 0, n)),
        ],
        out_specs=pl.BlockSpec((1, 16, MN), lambda b, n: (b, 0, n)),
        out_shape=jax.ShapeDtypeStruct((p_pad8.shape[0], 16, N), jnp.int32),
        scratch_shapes=[pltpu.VMEM((N, MN), jnp.float32)],
        compiler_params=pltpu.CompilerParams(
            dimension_semantics=("arbitrary", "arbitrary")),
    )(p_pad8, pT8)


_NW = 32
_CH = 128


def _gather_body(rows_w, kv_hbm, pp_hbm, idx_hbm, nkv_hbm, np_hbm,
                 idxv, kvbuf, ppbuf, sem1, sem2):
    cid = lax.axis_index("c")
    sid = lax.axis_index("s")
    wid = sid * 2 + cid
    base = wid * rows_w

    def body(i, carry):
        off = base + i * _CH
        pltpu.sync_copy(idx_hbm.at[pl.ds(off, _CH)], idxv)
        cp1 = pltpu.async_copy(kv_hbm.at[idxv], kvbuf, sem1)
        cp2 = pltpu.async_copy(pp_hbm.at[idxv], ppbuf, sem2)
        cp1.wait()
        cp2.wait()
        pltpu.sync_copy(kvbuf, nkv_hbm.at[pl.ds(off, _CH)])
        pltpu.sync_copy(ppbuf, np_hbm.at[pl.ds(off, _CH)])
        return carry

    lax.fori_loop(0, rows_w // _CH, body, 0)


def _gather_call(kv_flat, pp_flat, idx_flat):
    rows = idx_flat.shape[0]
    mesh = plsc.VectorSubcoreMesh(core_axis_name="c", subcore_axis_name="s")
    fn = pl.kernel(
        functools.partial(_gather_body, rows // _NW),
        out_type=[
            jax.ShapeDtypeStruct((rows, 2 * C), jnp.float32),
            jax.ShapeDtypeStruct((rows, 16), jnp.float32),
        ],
        mesh=mesh,
        scratch_types=[
            pltpu.VMEM((_CH,), jnp.int32),
            pltpu.VMEM((_CH, 2 * C), jnp.float32),
            pltpu.VMEM((_CH, 16), jnp.float32),
            pltpu.SemaphoreType.DMA,
            pltpu.SemaphoreType.DMA,
        ],
        compiler_params=pltpu.CompilerParams(use_tc_tiling_on_sc=False),
    )
    return fn(kv_flat, pp_flat, idx_flat)


def _zstats_body(p16_ref, npp_ref, w1t_ref, sumz_ref, sumsq_ref):
    @pl.when(jnp.logical_and(pl.program_id(0) == 0, pl.program_id(1) == 0))
    def _():
        sumz_ref[...] = jnp.zeros_like(sumz_ref)
        sumsq_ref[...] = jnp.zeros_like(sumsq_ref)

    ssum = jnp.zeros((1, C), jnp.float32)
    ssq = jnp.zeros((1, C), jnp.float32)
    prow = p16_ref[0]
    for k in range(K):
        r_k = prow - npp_ref[0, k]
        z = jnp.dot(r_k, w1t_ref[...], preferred_element_type=jnp.float32)
        ssum = ssum + jnp.sum(z, axis=0, keepdims=True)
        ssq = ssq + jnp.sum(z * z, axis=0, keepdims=True)
    sumz_ref[...] += ssum
    sumsq_ref[...] += ssq


def _zstats_call(p16, npp, W1T_pad):
    return pl.pallas_call(
        _zstats_body,
        grid=(p16.shape[0], N // NB),
        in_specs=[
            pl.BlockSpec((1, NB, 16), lambda b, n: (b, n, 0)),
            pl.BlockSpec((1, K, NB, 16), lambda b, n: (b, 0, n, 0)),
            pl.BlockSpec((16, C), lambda b, n: (0, 0)),
        ],
        out_specs=[
            pl.BlockSpec((1, C), lambda b, n: (0, 0)),
            pl.BlockSpec((1, C), lambda b, n: (0, 0)),
        ],
        out_shape=[
            jax.ShapeDtypeStruct((1, C), jnp.float32),
            jax.ShapeDtypeStruct((1, C), jnp.float32),
        ],
        compiler_params=pltpu.CompilerParams(
            dimension_semantics=("arbitrary", "arbitrary")),
    )(p16, npp, W1T_pad)


def _p1_body(p16_ref, npp_ref, nkv_ref, qT_ref, w1t_ref, s1_ref, t1_ref,
             w2t_ref, c2_ref, apre_ref, nr_ref, sum2_ref, sumsq2_ref):
    @pl.when(jnp.logical_and(pl.program_id(0) == 0, pl.program_id(1) == 0))
    def _():
        sum2_ref[...] = jnp.zeros_like(sum2_ref)
        sumsq2_ref[...] = jnp.zeros_like(sumsq2_ref)

    prow = p16_ref[0]
    qrow = qT_ref[0]
    s1 = s1_ref[...]
    t1 = t1_ref[...]
    c2 = c2_ref[...]
    ssum = jnp.zeros((1, C), jnp.float32)
    ssq = jnp.zeros((1, C), jnp.float32)
    for k in range(K):
        r_k = prow - npp_ref[0, k]
        z = jnp.dot(r_k, w1t_ref[...], preferred_element_type=jnp.float32)
        h1 = jnp.maximum(z * s1 + t1, 0.0)
        nr_k = jnp.dot(h1, w2t_ref[...], preferred_element_type=jnp.float32) + c2
        apre_k = qrow - nkv_ref[0, k] + nr_k
        nr_ref[0, k] = nr_k
        apre_ref[0, k] = apre_k
        ssum = ssum + jnp.sum(apre_k, axis=0, keepdims=True)
        ssq = ssq + jnp.sum(apre_k * apre_k, axis=0, keepdims=True)
    sum2_ref[...] += ssum
    sumsq2_ref[...] += ssq


def _p1_call(p16, npp, nkv, qT, W1T_pad, s1, t1, W2T, c2):
    nb = p16.shape[0]
    return pl.pallas_call(
        _p1_body,
        grid=(nb, N // NB),
        in_specs=[
            pl.BlockSpec((1, NB, 16), lambda b, n: (b, n, 0)),
            pl.BlockSpec((1, K, NB, 16), lambda b, n: (b, 0, n, 0)),
            pl.BlockSpec((1, K, NB, C), lambda b, n: (b, 0, n, 0)),
            pl.BlockSpec((1, NB, C), lambda b, n: (b, n, 0)),
            pl.BlockSpec((16, C), lambda b, n: (0, 0)),
            pl.BlockSpec((1, C), lambda b, n: (0, 0)),
            pl.BlockSpec((1, C), lambda b, n: (0, 0)),
            pl.BlockSpec((C, C), lambda b, n: (0, 0)),
            pl.BlockSpec((1, C), lambda b, n: (0, 0)),
        ],
        out_specs=[
            pl.BlockSpec((1, K, NB, C), lambda b, n: (b, 0, n, 0)),
            pl.BlockSpec((1, K, NB, C), lambda b, n: (b, 0, n, 0)),
            pl.BlockSpec((1, C), lambda b, n: (0, 0)),
            pl.BlockSpec((1, C), lambda b, n: (0, 0)),
        ],
        out_shape=[
            jax.ShapeDtypeStruct((nb, K, N, C), jnp.float32),
            jax.ShapeDtypeStruct((nb, K, N, C), jnp.float32),
            jax.ShapeDtypeStruct((1, C), jnp.float32),
            jax.ShapeDtypeStruct((1, C), jnp.float32),
        ],
        compiler_params=pltpu.CompilerParams(
            dimension_semantics=("arbitrary", "arbitrary")),
    )(p16, npp, nkv, qT, W1T_pad, s1, t1, W2T, c2)


def _p2_body(apre_ref, s2_ref, t2_ref, w1t_ref, a1_ref, sumh_ref, g_ref):
    @pl.when(jnp.logical_and(pl.program_id(0) == 0, pl.program_id(1) == 0))
    def _():
        sumh_ref[...] = jnp.zeros_like(sumh_ref)
        g_ref[...] = jnp.zeros_like(g_ref)

    s2 = s2_ref[...]
    t2 = t2_ref[...]
    ssum = jnp.zeros((1, C), jnp.float32)
    gacc = jnp.zeros((C, C), jnp.float32)
    for k in range(K):
        h = jnp.maximum(apre_ref[0, k] * s2 + t2, 0.0)
        a1_ref[0, k] = jnp.dot(h, w1t_ref[...], preferred_element_type=jnp.float32)
        ssum = ssum + jnp.sum(h, axis=0, keepdims=True)
        gacc = gacc + jnp.dot(h.T, h, preferred_element_type=jnp.float32)
    sumh_ref[...] += ssum
    g_ref[...] += gacc


def _p2_call(apre, s2, t2, atW1T):
    nb = apre.shape[0]
    return pl.pallas_call(
        _p2_body,
        grid=(nb, N // NB),
        in_specs=[
            pl.BlockSpec((1, K, NB, C), lambda b, n: (b, 0, n, 0)),
            pl.BlockSpec((1, C), lambda b, n: (0, 0)),
            pl.BlockSpec((1, C), lambda b, n: (0, 0)),
            pl.BlockSpec((C, C), lambda b, n: (0, 0)),
        ],
        out_specs=[
            pl.BlockSpec((1, K, NB, C), lambda b, n: (b, 0, n, 0)),
            pl.BlockSpec((1, C), lambda b, n: (0, 0)),
            pl.BlockSpec((C, C), lambda b, n: (0, 0)),
        ],
        out_shape=[
            jax.ShapeDtypeStruct((nb, K, N, C), jnp.float32),
            jax.ShapeDtypeStruct((1, C), jnp.float32),
            jax.ShapeDtypeStruct((C, C), jnp.float32),
        ],
        compiler_params=pltpu.CompilerParams(
            dimension_semantics=("arbitrary", "arbitrary")),
    )(apre, s2, t2, atW1T)


def _p3_body(a1_ref, nr_ref, nkv_ref, s3_ref, t3_ref, w2t_ref, c2_ref, y_ref):
    s3 = s3_ref[...]
    t3 = t3_ref[...]
    c2 = c2_ref[...]
    a2 = []
    for k in range(K):
        h = jnp.maximum(a1_ref[0, k] * s3 + t3, 0.0)
        a2.append(jnp.dot(h, w2t_ref[...], preferred_element_type=jnp.float32) + c2)
    mx = a2[0]
    for k in range(1, K):
        mx = jnp.maximum(mx, a2[k])
    esum = jnp.zeros_like(mx)
    ynum = jnp.zeros_like(mx)
    for k in range(K):
        e = jnp.exp(a2[k] - mx)
        esum = esum + e
        ynum = ynum + e * (nkv_ref[0, k] + nr_ref[0, k])
    y = ynum / esum
    y_ref[0] = y.T


def _p3_call(a1, nr, nkv, s3, t3, atW2T, atc2):
    nb = a1.shape[0]
    return pl.pallas_call(
        _p3_body,
        grid=(nb, N // NB),
        in_specs=[
            pl.BlockSpec((1, K, NB, C), lambda b, n: (b, 0, n, 0)),
            pl.BlockSpec((1, K, NB, C), lambda b, n: (b, 0, n, 0)),
            pl.BlockSpec((1, K, NB, C), lambda b, n: (b, 0, n, 1)),
            pl.BlockSpec((1, C), lambda b, n: (0, 0)),
            pl.BlockSpec((1, C), lambda b, n: (0, 0)),
            pl.BlockSpec((C, C), lambda b, n: (0, 0)),
            pl.BlockSpec((1, C), lambda b, n: (0, 0)),
        ],
        out_specs=pl.BlockSpec((1, C, NB), lambda b, n: (b, 0, n)),
        out_shape=jax.ShapeDtypeStruct((nb, C, N), jnp.float32),
        compiler_params=pltpu.CompilerParams(
            dimension_semantics=("arbitrary", "arbitrary")),
    )(a1, nr, nkv, s3, t3, atW2T, atc2)


def _bn_coef(g, b, mean, var):
    s = g / jnp.sqrt(var + EPS)
    return (s[None, :], (b - mean * s)[None, :])


def kernel(p, x, Wq, bq, Wk, bk, Wv, bv, pe_W1, pe_g1, pe_b1, pe_W2, pe_c2,
           at_g1, at_b1, at_W1, at_g2, at_b2, at_W2, at_c2):
    f32 = jnp.float32
    p = p.astype(f32)
    p_pad8 = jnp.pad(p, ((0, 0), (0, 0), (0, 8 - CD)))
    pT8 = jnp.transpose(p_pad8, (0, 2, 1))
    p16 = jnp.pad(p, ((0, 0), (0, 0), (0, 16 - CD)))
    W1T_pad = jnp.pad(pe_W1.T, ((0, 16 - CD), (0, 0)))

    qT, kvT = _qkv_call(x, Wq.T, Wk.T, Wv.T, bq[None, :], bk[None, :], bv[None, :])
    kv_flat = kvT.reshape(B * N, 2 * C)
    pp_flat = p16.reshape(B * N, 16)
    BH = B // 2

    idxT_h = [_topk_call(p_pad8[h * BH:(h + 1) * BH],
                         pT8[h * BH:(h + 1) * BH], h * BH) for h in range(2)]
    g_h = [_gather_call(kv_flat, pp_flat, idxT_h[h].reshape(-1)) for h in range(2)]
    nkv_h = [g[0].reshape(BH, K, N, 2 * C) for g in g_h]
    npp_h = [g[1].reshape(BH, K, N, 16) for g in g_h]
    p16_h = [p16[h * BH:(h + 1) * BH] for h in range(2)]
    qT_h = [qT[h * BH:(h + 1) * BH] for h in range(2)]

    zs = [_zstats_call(p16_h[h], npp_h[h], W1T_pad) for h in range(2)]
    mean1 = (zs[0][0][0] + zs[1][0][0]) / BNK
    var1 = (zs[0][1][0] + zs[1][1][0]) / BNK - mean1 * mean1
    s1, t1 = _bn_coef(pe_g1, pe_b1, mean1, var1)

    p1 = [_p1_call(p16_h[h], npp_h[h], nkv_h[h], qT_h[h],
                   W1T_pad, s1, t1, pe_W2.T, pe_c2[None, :]) for h in range(2)]
    mean2 = (p1[0][2][0] + p1[1][2][0]) / BNK
    var2 = (p1[0][3][0] + p1[1][3][0]) / BNK - mean2 * mean2
    s2, t2 = _bn_coef(at_g1, at_b1, mean2, var2)

    p2 = [_p2_call(p1[h][0], s2, t2, at_W1.T) for h in range(2)]
    mu_h = (p2[0][1][0] + p2[1][1][0]) / BNK
    G = p2[0][2] + p2[1][2]
    mean3 = at_W1 @ mu_h
    e2 = jnp.sum((at_W1 @ G) * at_W1, axis=1) / BNK
    var3 = e2 - mean3 * mean3
    s3, t3 = _bn_coef(at_g2, at_b2, mean3, var3)

    y_h = [_p3_call(p2[h][0], p1[h][1], nkv_h[h], s3, t3,
                    at_W2.T, at_c2[None, :]) for h in range(2)]
    y = jnp.concatenate(y_h, axis=0)
    return (p, y)


# speedup vs baseline: 11.1714x; 1.0088x over previous
import functools

import jax
import jax.numpy as jnp
from jax import lax
from jax.experimental import pallas as pl
from jax.experimental.pallas import tpu as pltpu
from jax.experimental.pallas import tpu_sc as plsc

B, N, C, K, CD = 4, 4096, 128, 16, 3
MN = 512
NB = 512
BIG = 1e30
EPS = 1e-5
BNK = B * N * K


def _qkv_body(x_ref, wqt_ref, wkt_ref, wvt_ref, bq_ref, bk_ref, bv_ref,
              qT_ref, kvT_ref):
    xt = x_ref[0].T
    qT_ref[0] = jnp.dot(xt, wqt_ref[...], preferred_element_type=jnp.float32) + bq_ref[...]
    kvT_ref[0, :, 0:C] = jnp.dot(xt, wkt_ref[...], preferred_element_type=jnp.float32) + bk_ref[...]
    kvT_ref[0, :, C:2 * C] = jnp.dot(xt, wvt_ref[...], preferred_element_type=jnp.float32) + bv_ref[...]


def _qkv_call(x, WqT, WkT, WvT, bq, bk, bv):
    return pl.pallas_call(
        _qkv_body,
        grid=(B, N // NB),
        in_specs=[
            pl.BlockSpec((1, C, NB), lambda b, n: (b, 0, n)),
            pl.BlockSpec((C, C), lambda b, n: (0, 0)),
            pl.BlockSpec((C, C), lambda b, n: (0, 0)),
            pl.BlockSpec((C, C), lambda b, n: (0, 0)),
            pl.BlockSpec((1, C), lambda b, n: (0, 0)),
            pl.BlockSpec((1, C), lambda b, n: (0, 0)),
            pl.BlockSpec((1, C), lambda b, n: (0, 0)),
        ],
        out_specs=[
            pl.BlockSpec((1, NB, C), lambda b, n: (b, n, 0)),
            pl.BlockSpec((1, NB, 2 * C), lambda b, n: (b, n, 0)),
        ],
        out_shape=[
            jax.ShapeDtypeStruct((B, N, C), jnp.float32),
            jax.ShapeDtypeStruct((B, N, 2 * C), jnp.float32),
        ],
        compiler_params=pltpu.CompilerParams(
            dimension_semantics=("arbitrary", "arbitrary")),
    )(x, WqT, WkT, WvT, bq, bk, bv)


def _topk_body(boff, pa_ref, prt_ref, idx_ref, dist_ref):
    b = pl.program_id(0) + boff
    pa = pa_ref[0]
    prt = prt_ref[0]
    sq_all = jnp.sum(pa * pa, axis=1, keepdims=True)
    sq_rows = jnp.sum(prt * prt, axis=0, keepdims=True)
    inner = jnp.dot(pa, prt, preferred_element_type=jnp.float32)
    d = jnp.maximum(sq_all - 2.0 * inner + sq_rows, 0.0)
    zmask = jnp.all(pa == 0.0, axis=1, keepdims=True)
    dist_ref[...] = jnp.where(zmask, BIG, d)
    iota0 = lax.broadcasted_iota(jnp.int32, (N, MN), 0)
    rows16 = lax.broadcasted_iota(jnp.int32, (16, MN), 0)

    def step(kk, acc):
        dd = dist_ref[...]
        m = jnp.min(dd, axis=0, keepdims=True)
        sel = jnp.where(dd == m, iota0, N)
        j = jnp.min(sel, axis=0, keepdims=True)
        dist_ref[...] = jnp.where(iota0 == j, BIG, dd)
        return jnp.where(rows16 == kk, j + b * N, acc)

    idx_ref[0] = lax.fori_loop(0, 16, step, jnp.zeros((16, MN), jnp.int32))


def _topk_call(p_pad8, pT8, boff):
    nb = p_pad8.shape[0]
    return pl.pallas_call(
        functools.partial(_topk_body, boff),
        grid=(nb, N // MN),
        in_specs=[
            pl.BlockSpec((1, N, 8), lambda b, n: (b, 0, 0)),
            pl.BlockSpec((1, 8, MN), lambda b, n: (b, 0, n)),
        ],
        out_specs=pl.BlockSpec((1, 16, MN), lambda b, n: (b, 0, n)),
        out_shape=jax.ShapeDtypeStruct((p_pad8.shape[0], 16, N), jnp.int32),
        scratch_shapes=[pltpu.VMEM((N, MN), jnp.float32)],
        compiler_params=pltpu.CompilerParams(
            dimension_semantics=("arbitrary", "arbitrary")),
    )(p_pad8, pT8)


_NW = 32
_CH = 128


def _gather_body(rows_w, kv_hbm, pp_hbm, idx_hbm, nkv_hbm, np_hbm,
                 idxv, kvbuf, ppbuf, sem1, sem2):
    cid = lax.axis_index("c")
    sid = lax.axis_index("s")
    wid = sid * 2 + cid
    base = wid * rows_w

    def body(i, carry):
        off = base + i * _CH
        pltpu.sync_copy(idx_hbm.at[pl.ds(off, _CH)], idxv)
        cp1 = pltpu.async_copy(kv_hbm.at[idxv], kvbuf, sem1)
        cp2 = pltpu.async_copy(pp_hbm.at[idxv], ppbuf, sem2)
        cp1.wait()
        cp2.wait()
        pltpu.sync_copy(kvbuf, nkv_hbm.at[pl.ds(off, _CH)])
        pltpu.sync_copy(ppbuf, np_hbm.at[pl.ds(off, _CH)])
        return carry

    lax.fori_loop(0, rows_w // _CH, body, 0)


def _gather_call(kv_flat, pp_flat, idx_flat):
    rows = idx_flat.shape[0]
    mesh = plsc.VectorSubcoreMesh(core_axis_name="c", subcore_axis_name="s")
    fn = pl.kernel(
        functools.partial(_gather_body, rows // _NW),
        out_type=[
            jax.ShapeDtypeStruct((rows, 2 * C), jnp.float32),
            jax.ShapeDtypeStruct((rows, 16), jnp.float32),
        ],
        mesh=mesh,
        scratch_types=[
            pltpu.VMEM((_CH,), jnp.int32),
            pltpu.VMEM((_CH, 2 * C), jnp.float32),
            pltpu.VMEM((_CH, 16), jnp.float32),
            pltpu.SemaphoreType.DMA,
            pltpu.SemaphoreType.DMA,
        ],
        compiler_params=pltpu.CompilerParams(use_tc_tiling_on_sc=False),
    )
    return fn(kv_flat, pp_flat, idx_flat)


def _zstats_body(p16_ref, npp_ref, w1t_ref, sumz_ref, sumsq_ref):
    @pl.when(jnp.logical_and(pl.program_id(0) == 0, pl.program_id(1) == 0))
    def _():
        sumz_ref[...] = jnp.zeros_like(sumz_ref)
        sumsq_ref[...] = jnp.zeros_like(sumsq_ref)

    ssum = jnp.zeros((1, C), jnp.float32)
    ssq = jnp.zeros((1, C), jnp.float32)
    prow = p16_ref[0]
    for k in range(K):
        r_k = prow - npp_ref[0, k]
        z = jnp.dot(r_k, w1t_ref[...], preferred_element_type=jnp.float32)
        ssum = ssum + jnp.sum(z, axis=0, keepdims=True)
        ssq = ssq + jnp.sum(z * z, axis=0, keepdims=True)
    sumz_ref[...] += ssum
    sumsq_ref[...] += ssq


def _zstats_call(p16, npp, W1T_pad):
    return pl.pallas_call(
        _zstats_body,
        grid=(p16.shape[0], N // NB),
        in_specs=[
            pl.BlockSpec((1, NB, 16), lambda b, n: (b, n, 0)),
            pl.BlockSpec((1, K, NB, 16), lambda b, n: (b, 0, n, 0)),
            pl.BlockSpec((16, C), lambda b, n: (0, 0)),
        ],
        out_specs=[
            pl.BlockSpec((1, C), lambda b, n: (0, 0)),
            pl.BlockSpec((1, C), lambda b, n: (0, 0)),
        ],
        out_shape=[
            jax.ShapeDtypeStruct((1, C), jnp.float32),
            jax.ShapeDtypeStruct((1, C), jnp.float32),
        ],
        compiler_params=pltpu.CompilerParams(
            dimension_semantics=("arbitrary", "arbitrary")),
    )(p16, npp, W1T_pad)


def _p1_body(p16_ref, npp_ref, nkv_ref, qT_ref, w1t_ref, s1_ref, t1_ref,
             w2t_ref, c2_ref, apre_ref, nr_ref, sum2_ref, sumsq2_ref):
    @pl.when(jnp.logical_and(pl.program_id(0) == 0, pl.program_id(1) == 0))
    def _():
        sum2_ref[...] = jnp.zeros_like(sum2_ref)
        sumsq2_ref[...] = jnp.zeros_like(sumsq2_ref)

    prow = p16_ref[0]
    qrow = qT_ref[0]
    s1 = s1_ref[...]
    t1 = t1_ref[...]
    c2 = c2_ref[...]
    ssum = jnp.zeros((1, C), jnp.float32)
    ssq = jnp.zeros((1, C), jnp.float32)
    for k in range(K):
        r_k = prow - npp_ref[0, k]
        z = jnp.dot(r_k, w1t_ref[...], preferred_element_type=jnp.float32)
        h1 = jnp.maximum(z * s1 + t1, 0.0)
        nr_k = jnp.dot(h1, w2t_ref[...], preferred_element_type=jnp.float32) + c2
        apre_k = qrow - nkv_ref[0, k] + nr_k
        nr_ref[0, k] = nr_k
        apre_ref[0, k] = apre_k
        ssum = ssum + jnp.sum(apre_k, axis=0, keepdims=True)
        ssq = ssq + jnp.sum(apre_k * apre_k, axis=0, keepdims=True)
    sum2_ref[...] += ssum
    sumsq2_ref[...] += ssq


def _p1_call(p16, npp, nkv, qT, W1T_pad, s1, t1, W2T, c2):
    nb = p16.shape[0]
    return pl.pallas_call(
        _p1_body,
        grid=(nb, N // NB),
        in_specs=[
            pl.BlockSpec((1, NB, 16), lambda b, n: (b, n, 0)),
            pl.BlockSpec((1, K, NB, 16), lambda b, n: (b, 0, n, 0)),
            pl.BlockSpec((1, K, NB, C), lambda b, n: (b, 0, n, 0)),
            pl.BlockSpec((1, NB, C), lambda b, n: (b, n, 0)),
            pl.BlockSpec((16, C), lambda b, n: (0, 0)),
            pl.BlockSpec((1, C), lambda b, n: (0, 0)),
            pl.BlockSpec((1, C), lambda b, n: (0, 0)),
            pl.BlockSpec((C, C), lambda b, n: (0, 0)),
            pl.BlockSpec((1, C), lambda b, n: (0, 0)),
        ],
        out_specs=[
            pl.BlockSpec((1, K, NB, C), lambda b, n: (b, 0, n, 0)),
            pl.BlockSpec((1, K, NB, C), lambda b, n: (b, 0, n, 0)),
            pl.BlockSpec((1, C), lambda b, n: (0, 0)),
            pl.BlockSpec((1, C), lambda b, n: (0, 0)),
        ],
        out_shape=[
            jax.ShapeDtypeStruct((nb, K, N, C), jnp.float32),
            jax.ShapeDtypeStruct((nb, K, N, C), jnp.float32),
            jax.ShapeDtypeStruct((1, C), jnp.float32),
            jax.ShapeDtypeStruct((1, C), jnp.float32),
        ],
        compiler_params=pltpu.CompilerParams(
            dimension_semantics=("arbitrary", "arbitrary")),
    )(p16, npp, nkv, qT, W1T_pad, s1, t1, W2T, c2)


def _p2_body(apre_ref, s2_ref, t2_ref, w1t_ref, a1_ref, sumh_ref, g_ref):
    @pl.when(jnp.logical_and(pl.program_id(0) == 0, pl.program_id(1) == 0))
    def _():
        sumh_ref[...] = jnp.zeros_like(sumh_ref)
        g_ref[...] = jnp.zeros_like(g_ref)

    s2 = s2_ref[...]
    t2 = t2_ref[...]
    ssum = jnp.zeros((1, C), jnp.float32)
    gacc = jnp.zeros((C, C), jnp.float32)
    for k in range(K):
        h = jnp.maximum(apre_ref[0, k] * s2 + t2, 0.0)
        a1_ref[0, k] = jnp.dot(h, w1t_ref[...], preferred_element_type=jnp.float32)
        ssum = ssum + jnp.sum(h, axis=0, keepdims=True)
        gacc = gacc + jnp.dot(h.T, h, preferred_element_type=jnp.float32)
    sumh_ref[...] += ssum
    g_ref[...] += gacc


def _p2_call(apre, s2, t2, atW1T):
    nb = apre.shape[0]
    return pl.pallas_call(
        _p2_body,
        grid=(nb, N // NB),
        in_specs=[
            pl.BlockSpec((1, K, NB, C), lambda b, n: (b, 0, n, 0)),
            pl.BlockSpec((1, C), lambda b, n: (0, 0)),
            pl.BlockSpec((1, C), lambda b, n: (0, 0)),
            pl.BlockSpec((C, C), lambda b, n: (0, 0)),
        ],
        out_specs=[
            pl.BlockSpec((1, K, NB, C), lambda b, n: (b, 0, n, 0)),
            pl.BlockSpec((1, C), lambda b, n: (0, 0)),
            pl.BlockSpec((C, C), lambda b, n: (0, 0)),
        ],
        out_shape=[
            jax.ShapeDtypeStruct((nb, K, N, C), jnp.float32),
            jax.ShapeDtypeStruct((1, C), jnp.float32),
            jax.ShapeDtypeStruct((C, C), jnp.float32),
        ],
        compiler_params=pltpu.CompilerParams(
            dimension_semantics=("arbitrary", "arbitrary")),
    )(apre, s2, t2, atW1T)


def _p3_body(a1_ref, nr_ref, nkv_ref, s3_ref, t3_ref, w2t_ref, c2_ref, y_ref):
    s3 = s3_ref[...]
    t3 = t3_ref[...]
    c2 = c2_ref[...]
    a2 = []
    for k in range(K):
        h = jnp.maximum(a1_ref[0, k] * s3 + t3, 0.0)
        a2.append(jnp.dot(h, w2t_ref[...], preferred_element_type=jnp.float32) + c2)
    mx = a2[0]
    for k in range(1, K):
        mx = jnp.maximum(mx, a2[k])
    esum = jnp.zeros_like(mx)
    ynum = jnp.zeros_like(mx)
    for k in range(K):
        e = jnp.exp(a2[k] - mx)
        esum = esum + e
        ynum = ynum + e * (nkv_ref[0, k] + nr_ref[0, k])
    y = ynum / esum
    y_ref[0] = y.T


def _p3_call(a1, nr, nkv, s3, t3, atW2T, atc2):
    nb = a1.shape[0]
    return pl.pallas_call(
        _p3_body,
        grid=(nb, N // NB),
        in_specs=[
            pl.BlockSpec((1, K, NB, C), lambda b, n: (b, 0, n, 0)),
            pl.BlockSpec((1, K, NB, C), lambda b, n: (b, 0, n, 0)),
            pl.BlockSpec((1, K, NB, C), lambda b, n: (b, 0, n, 1)),
            pl.BlockSpec((1, C), lambda b, n: (0, 0)),
            pl.BlockSpec((1, C), lambda b, n: (0, 0)),
            pl.BlockSpec((C, C), lambda b, n: (0, 0)),
            pl.BlockSpec((1, C), lambda b, n: (0, 0)),
        ],
        out_specs=pl.BlockSpec((1, C, NB), lambda b, n: (b, 0, n)),
        out_shape=jax.ShapeDtypeStruct((nb, C, N), jnp.float32),
        compiler_params=pltpu.CompilerParams(
            dimension_semantics=("arbitrary", "arbitrary")),
    )(a1, nr, nkv, s3, t3, atW2T, atc2)


def _bn_coef(g, b, mean, var):
    s = g / jnp.sqrt(var + EPS)
    return (s[None, :], (b - mean * s)[None, :])


def kernel(p, x, Wq, bq, Wk, bk, Wv, bv, pe_W1, pe_g1, pe_b1, pe_W2, pe_c2,
           at_g1, at_b1, at_W1, at_g2, at_b2, at_W2, at_c2):
    f32 = jnp.float32
    p = p.astype(f32)
    p_pad8 = jnp.pad(p, ((0, 0), (0, 0), (0, 8 - CD)))
    pT8 = jnp.transpose(p_pad8, (0, 2, 1))
    p16 = jnp.pad(p, ((0, 0), (0, 0), (0, 16 - CD)))
    W1T_pad = jnp.pad(pe_W1.T, ((0, 16 - CD), (0, 0)))

    qT, kvT = _qkv_call(x, Wq.T, Wk.T, Wv.T, bq[None, :], bk[None, :], bv[None, :])
    kv_flat = kvT.reshape(B * N, 2 * C)
    pp_flat = p16.reshape(B * N, 16)
    BH = 1

    idxT_h = [_topk_call(p_pad8[h * BH:(h + 1) * BH],
                         pT8[h * BH:(h + 1) * BH], h * BH) for h in range(B // BH)]
    g_h = [_gather_call(kv_flat, pp_flat, idxT_h[h].reshape(-1)) for h in range(B // BH)]
    nkv_h = [g[0].reshape(BH, K, N, 2 * C) for g in g_h]
    npp_h = [g[1].reshape(BH, K, N, 16) for g in g_h]
    p16_h = [p16[h * BH:(h + 1) * BH] for h in range(B // BH)]
    qT_h = [qT[h * BH:(h + 1) * BH] for h in range(B // BH)]

    zs = [_zstats_call(p16_h[h], npp_h[h], W1T_pad) for h in range(B // BH)]
    mean1 = sum(z[0][0] for z in zs) / BNK
    var1 = sum(z[1][0] for z in zs) / BNK - mean1 * mean1
    s1, t1 = _bn_coef(pe_g1, pe_b1, mean1, var1)

    p1 = [_p1_call(p16_h[h], npp_h[h], nkv_h[h], qT_h[h],
                   W1T_pad, s1, t1, pe_W2.T, pe_c2[None, :]) for h in range(B // BH)]
    mean2 = sum(q[2][0] for q in p1) / BNK
    var2 = sum(q[3][0] for q in p1) / BNK - mean2 * mean2
    s2, t2 = _bn_coef(at_g1, at_b1, mean2, var2)

    p2 = [_p2_call(p1[h][0], s2, t2, at_W1.T) for h in range(B // BH)]
    mu_h = sum(q[1][0] for q in p2) / BNK
    G = sum(q[2] for q in p2)
    mean3 = at_W1 @ mu_h
    e2 = jnp.sum((at_W1 @ G) * at_W1, axis=1) / BNK
    var3 = e2 - mean3 * mean3
    s3, t3 = _bn_coef(at_g2, at_b2, mean3, var3)

    y_h = [_p3_call(p2[h][0], p1[h][1], nkv_h[h], s3, t3,
                    at_W2.T, at_c2[None, :]) for h in range(B // BH)]
    y = jnp.concatenate(y_h, axis=0)
    return (p, y)


# speedup vs baseline: 12.0865x; 1.0819x over previous
import functools

import jax
import jax.numpy as jnp
from jax import lax
from jax.experimental import pallas as pl
from jax.experimental.pallas import tpu as pltpu
from jax.experimental.pallas import tpu_sc as plsc

B, N, C, K, CD = 4, 4096, 128, 16, 3
MN = 1024
NB = 512
BIG = 1e30
EPS = 1e-5
BNK = B * N * K


def _qkv_body(x_ref, wqt_ref, wkt_ref, wvt_ref, bq_ref, bk_ref, bv_ref,
              qT_ref, kvT_ref):
    xt = x_ref[0].T
    qT_ref[0] = jnp.dot(xt, wqt_ref[...], preferred_element_type=jnp.float32) + bq_ref[...]
    kvT_ref[0, :, 0:C] = jnp.dot(xt, wkt_ref[...], preferred_element_type=jnp.float32) + bk_ref[...]
    kvT_ref[0, :, C:2 * C] = jnp.dot(xt, wvt_ref[...], preferred_element_type=jnp.float32) + bv_ref[...]


def _qkv_call(x, WqT, WkT, WvT, bq, bk, bv):
    return pl.pallas_call(
        _qkv_body,
        grid=(B, N // NB),
        in_specs=[
            pl.BlockSpec((1, C, NB), lambda b, n: (b, 0, n)),
            pl.BlockSpec((C, C), lambda b, n: (0, 0)),
            pl.BlockSpec((C, C), lambda b, n: (0, 0)),
            pl.BlockSpec((C, C), lambda b, n: (0, 0)),
            pl.BlockSpec((1, C), lambda b, n: (0, 0)),
            pl.BlockSpec((1, C), lambda b, n: (0, 0)),
            pl.BlockSpec((1, C), lambda b, n: (0, 0)),
        ],
        out_specs=[
            pl.BlockSpec((1, NB, C), lambda b, n: (b, n, 0)),
            pl.BlockSpec((1, NB, 2 * C), lambda b, n: (b, n, 0)),
        ],
        out_shape=[
            jax.ShapeDtypeStruct((B, N, C), jnp.float32),
            jax.ShapeDtypeStruct((B, N, 2 * C), jnp.float32),
        ],
        compiler_params=pltpu.CompilerParams(
            dimension_semantics=("arbitrary", "arbitrary")),
    )(x, WqT, WkT, WvT, bq, bk, bv)


def _topk_body(boff, pa_ref, prt_ref, idx_ref, dist_ref):
    b = pl.program_id(0) + boff
    pa = pa_ref[0]
    prt = prt_ref[0]
    sq_all = jnp.sum(pa * pa, axis=1, keepdims=True)
    sq_rows = jnp.sum(prt * prt, axis=0, keepdims=True)
    inner = jnp.dot(pa, prt, preferred_element_type=jnp.float32)
    d = jnp.maximum(sq_all - 2.0 * inner + sq_rows, 0.0)
    zmask = jnp.all(pa == 0.0, axis=1, keepdims=True)
    dist_ref[...] = jnp.where(zmask, BIG, d)
    iota0 = lax.broadcasted_iota(jnp.int32, (N, MN), 0)
    rows16 = lax.broadcasted_iota(jnp.int32, (16, MN), 0)

    def step(kk, acc):
        dd = dist_ref[...]
        m = jnp.min(dd, axis=0, keepdims=True)
        sel = jnp.where(dd == m, iota0, N)
        j = jnp.min(sel, axis=0, keepdims=True)
        dist_ref[...] = jnp.where(iota0 == j, BIG, dd)
        return jnp.where(rows16 == kk, j + b * N, acc)

    idx_ref[0] = lax.fori_loop(0, 16, step, jnp.zeros((16, MN), jnp.int32))


def _topk_call(p_pad8, pT8, boff):
    nb = p_pad8.shape[0]
    return pl.pallas_call(
        functools.partial(_topk_body, boff),
        grid=(nb, N // MN),
        in_specs=[
            pl.BlockSpec((1, N, 8), lambda b, n: (b, 0, 0)),
            pl.BlockSpec((1, 8, MN), lambda b, n: (b, 0, n)),
        ],
        out_specs=pl.BlockSpec((1, 16, MN), lambda b, n: (b, 0, n)),
        out_shape=jax.ShapeDtypeStruct((p_pad8.shape[0], 16, N), jnp.int32),
        scratch_shapes=[pltpu.VMEM((N, MN), jnp.float32)],
        compiler_params=pltpu.CompilerParams(
            dimension_semantics=("arbitrary", "arbitrary")),
    )(p_pad8, pT8)


_NW = 32
_CH = 128


def _gather_body(rows_w, kv_hbm, pp_hbm, idx_hbm, nkv_hbm, np_hbm,
                 idxv, kvbuf, ppbuf, sem1, sem2):
    cid = lax.axis_index("c")
    sid = lax.axis_index("s")
    wid = sid * 2 + cid
    base = wid * rows_w

    def body(i, carry):
        off = base + i * _CH
        pltpu.sync_copy(idx_hbm.at[pl.ds(off, _CH)], idxv)
        cp1 = pltpu.async_copy(kv_hbm.at[idxv], kvbuf, sem1)
        cp2 = pltpu.async_copy(pp_hbm.at[idxv], ppbuf, sem2)
        cp1.wait()
        cp2.wait()
        pltpu.sync_copy(kvbuf, nkv_hbm.at[pl.ds(off, _CH)])
        pltpu.sync_copy(ppbuf, np_hbm.at[pl.ds(off, _CH)])
        return carry

    lax.fori_loop(0, rows_w // _CH, body, 0)


def _gather_call(kv_flat, pp_flat, idx_flat):
    rows = idx_flat.shape[0]
    mesh = plsc.VectorSubcoreMesh(core_axis_name="c", subcore_axis_name="s")
    fn = pl.kernel(
        functools.partial(_gather_body, rows // _NW),
        out_type=[
            jax.ShapeDtypeStruct((rows, 2 * C), jnp.float32),
            jax.ShapeDtypeStruct((rows, 16), jnp.float32),
        ],
        mesh=mesh,
        scratch_types=[
            pltpu.VMEM((_CH,), jnp.int32),
            pltpu.VMEM((_CH, 2 * C), jnp.float32),
            pltpu.VMEM((_CH, 16), jnp.float32),
            pltpu.SemaphoreType.DMA,
            pltpu.SemaphoreType.DMA,
        ],
        compiler_params=pltpu.CompilerParams(use_tc_tiling_on_sc=False),
    )
    return fn(kv_flat, pp_flat, idx_flat)


def _zstats_body(p16_ref, npp_ref, w1t_ref, sumz_ref, sumsq_ref):
    @pl.when(jnp.logical_and(pl.program_id(0) == 0, pl.program_id(1) == 0))
    def _():
        sumz_ref[...] = jnp.zeros_like(sumz_ref)
        sumsq_ref[...] = jnp.zeros_like(sumsq_ref)

    ssum = jnp.zeros((1, C), jnp.float32)
    ssq = jnp.zeros((1, C), jnp.float32)
    prow = p16_ref[0]
    for k in range(K):
        r_k = prow - npp_ref[0, k]
        z = jnp.dot(r_k, w1t_ref[...], preferred_element_type=jnp.float32)
        ssum = ssum + jnp.sum(z, axis=0, keepdims=True)
        ssq = ssq + jnp.sum(z * z, axis=0, keepdims=True)
    sumz_ref[...] += ssum
    sumsq_ref[...] += ssq


def _zstats_call(p16, npp, W1T_pad):
    return pl.pallas_call(
        _zstats_body,
        grid=(p16.shape[0], N // NB),
        in_specs=[
            pl.BlockSpec((1, NB, 16), lambda b, n: (b, n, 0)),
            pl.BlockSpec((1, K, NB, 16), lambda b, n: (b, 0, n, 0)),
            pl.BlockSpec((16, C), lambda b, n: (0, 0)),
        ],
        out_specs=[
            pl.BlockSpec((1, C), lambda b, n: (0, 0)),
            pl.BlockSpec((1, C), lambda b, n: (0, 0)),
        ],
        out_shape=[
            jax.ShapeDtypeStruct((1, C), jnp.float32),
            jax.ShapeDtypeStruct((1, C), jnp.float32),
        ],
        compiler_params=pltpu.CompilerParams(
            dimension_semantics=("arbitrary", "arbitrary")),
    )(p16, npp, W1T_pad)


def _p1_body(p16_ref, npp_ref, nkv_ref, qT_ref, w1t_ref, s1_ref, t1_ref,
             w2t_ref, c2_ref, apre_ref, nr_ref, sum2_ref, sumsq2_ref):
    @pl.when(jnp.logical_and(pl.program_id(0) == 0, pl.program_id(1) == 0))
    def _():
        sum2_ref[...] = jnp.zeros_like(sum2_ref)
        sumsq2_ref[...] = jnp.zeros_like(sumsq2_ref)

    prow = p16_ref[0]
    qrow = qT_ref[0]
    s1 = s1_ref[...]
    t1 = t1_ref[...]
    c2 = c2_ref[...]
    ssum = jnp.zeros((1, C), jnp.float32)
    ssq = jnp.zeros((1, C), jnp.float32)
    for k in range(K):
        r_k = prow - npp_ref[0, k]
        z = jnp.dot(r_k, w1t_ref[...], preferred_element_type=jnp.float32)
        h1 = jnp.maximum(z * s1 + t1, 0.0)
        nr_k = jnp.dot(h1, w2t_ref[...], preferred_element_type=jnp.float32) + c2
        apre_k = qrow - nkv_ref[0, k] + nr_k
        nr_ref[0, k] = nr_k
        apre_ref[0, k] = apre_k
        ssum = ssum + jnp.sum(apre_k, axis=0, keepdims=True)
        ssq = ssq + jnp.sum(apre_k * apre_k, axis=0, keepdims=True)
    sum2_ref[...] += ssum
    sumsq2_ref[...] += ssq


def _p1_call(p16, npp, nkv, qT, W1T_pad, s1, t1, W2T, c2):
    nb = p16.shape[0]
    return pl.pallas_call(
        _p1_body,
        grid=(nb, N // NB),
        in_specs=[
            pl.BlockSpec((1, NB, 16), lambda b, n: (b, n, 0)),
            pl.BlockSpec((1, K, NB, 16), lambda b, n: (b, 0, n, 0)),
            pl.BlockSpec((1, K, NB, C), lambda b, n: (b, 0, n, 0)),
            pl.BlockSpec((1, NB, C), lambda b, n: (b, n, 0)),
            pl.BlockSpec((16, C), lambda b, n: (0, 0)),
            pl.BlockSpec((1, C), lambda b, n: (0, 0)),
            pl.BlockSpec((1, C), lambda b, n: (0, 0)),
            pl.BlockSpec((C, C), lambda b, n: (0, 0)),
            pl.BlockSpec((1, C), lambda b, n: (0, 0)),
        ],
        out_specs=[
            pl.BlockSpec((1, K, NB, C), lambda b, n: (b, 0, n, 0)),
            pl.BlockSpec((1, K, NB, C), lambda b, n: (b, 0, n, 0)),
            pl.BlockSpec((1, C), lambda b, n: (0, 0)),
            pl.BlockSpec((1, C), lambda b, n: (0, 0)),
        ],
        out_shape=[
            jax.ShapeDtypeStruct((nb, K, N, C), jnp.float32),
            jax.ShapeDtypeStruct((nb, K, N, C), jnp.float32),
            jax.ShapeDtypeStruct((1, C), jnp.float32),
            jax.ShapeDtypeStruct((1, C), jnp.float32),
        ],
        compiler_params=pltpu.CompilerParams(
            dimension_semantics=("arbitrary", "arbitrary")),
    )(p16, npp, nkv, qT, W1T_pad, s1, t1, W2T, c2)


def _p2_body(apre_ref, s2_ref, t2_ref, w1t_ref, a1_ref, sumh_ref, g_ref):
    @pl.when(jnp.logical_and(pl.program_id(0) == 0, pl.program_id(1) == 0))
    def _():
        sumh_ref[...] = jnp.zeros_like(sumh_ref)
        g_ref[...] = jnp.zeros_like(g_ref)

    s2 = s2_ref[...]
    t2 = t2_ref[...]
    ssum = jnp.zeros((1, C), jnp.float32)
    gacc = jnp.zeros((C, C), jnp.float32)
    for k in range(K):
        h = jnp.maximum(apre_ref[0, k] * s2 + t2, 0.0)
        a1_ref[0, k] = jnp.dot(h, w1t_ref[...], preferred_element_type=jnp.float32)
        ssum = ssum + jnp.sum(h, axis=0, keepdims=True)
        gacc = gacc + jnp.dot(h.T, h, preferred_element_type=jnp.float32)
    sumh_ref[...] += ssum
    g_ref[...] += gacc


def _p2_call(apre, s2, t2, atW1T):
    nb = apre.shape[0]
    return pl.pallas_call(
        _p2_body,
        grid=(nb, N // NB),
        in_specs=[
            pl.BlockSpec((1, K, NB, C), lambda b, n: (b, 0, n, 0)),
            pl.BlockSpec((1, C), lambda b, n: (0, 0)),
            pl.BlockSpec((1, C), lambda b, n: (0, 0)),
            pl.BlockSpec((C, C), lambda b, n: (0, 0)),
        ],
        out_specs=[
            pl.BlockSpec((1, K, NB, C), lambda b, n: (b, 0, n, 0)),
            pl.BlockSpec((1, C), lambda b, n: (0, 0)),
            pl.BlockSpec((C, C), lambda b, n: (0, 0)),
        ],
        out_shape=[
            jax.ShapeDtypeStruct((nb, K, N, C), jnp.float32),
            jax.ShapeDtypeStruct((1, C), jnp.float32),
            jax.ShapeDtypeStruct((C, C), jnp.float32),
        ],
        compiler_params=pltpu.CompilerParams(
            dimension_semantics=("arbitrary", "arbitrary")),
    )(apre, s2, t2, atW1T)


def _p3_body(a1_ref, nr_ref, nkv_ref, s3_ref, t3_ref, w2t_ref, c2_ref, y_ref):
    s3 = s3_ref[...]
    t3 = t3_ref[...]
    c2 = c2_ref[...]
    a2 = []
    for k in range(K):
        h = jnp.maximum(a1_ref[0, k] * s3 + t3, 0.0)
        a2.append(jnp.dot(h, w2t_ref[...], preferred_element_type=jnp.float32) + c2)
    mx = a2[0]
    for k in range(1, K):
        mx = jnp.maximum(mx, a2[k])
    esum = jnp.zeros_like(mx)
    ynum = jnp.zeros_like(mx)
    for k in range(K):
        e = jnp.exp(a2[k] - mx)
        esum = esum + e
        ynum = ynum + e * (nkv_ref[0, k] + nr_ref[0, k])
    y = ynum / esum
    y_ref[0] = y.T


def _p3_call(a1, nr, nkv, s3, t3, atW2T, atc2):
    nb = a1.shape[0]
    return pl.pallas_call(
        _p3_body,
        grid=(nb, N // NB),
        in_specs=[
            pl.BlockSpec((1, K, NB, C), lambda b, n: (b, 0, n, 0)),
            pl.BlockSpec((1, K, NB, C), lambda b, n: (b, 0, n, 0)),
            pl.BlockSpec((1, K, NB, C), lambda b, n: (b, 0, n, 1)),
            pl.BlockSpec((1, C), lambda b, n: (0, 0)),
            pl.BlockSpec((1, C), lambda b, n: (0, 0)),
            pl.BlockSpec((C, C), lambda b, n: (0, 0)),
            pl.BlockSpec((1, C), lambda b, n: (0, 0)),
        ],
        out_specs=pl.BlockSpec((1, C, NB), lambda b, n: (b, 0, n)),
        out_shape=jax.ShapeDtypeStruct((nb, C, N), jnp.float32),
        compiler_params=pltpu.CompilerParams(
            dimension_semantics=("arbitrary", "arbitrary")),
    )(a1, nr, nkv, s3, t3, atW2T, atc2)


def _bn_coef(g, b, mean, var):
    s = g / jnp.sqrt(var + EPS)
    return (s[None, :], (b - mean * s)[None, :])


def kernel(p, x, Wq, bq, Wk, bk, Wv, bv, pe_W1, pe_g1, pe_b1, pe_W2, pe_c2,
           at_g1, at_b1, at_W1, at_g2, at_b2, at_W2, at_c2):
    f32 = jnp.float32
    p = p.astype(f32)
    p_pad8 = jnp.pad(p, ((0, 0), (0, 0), (0, 8 - CD)))
    pT8 = jnp.transpose(p_pad8, (0, 2, 1))
    p16 = jnp.pad(p, ((0, 0), (0, 0), (0, 16 - CD)))
    W1T_pad = jnp.pad(pe_W1.T, ((0, 16 - CD), (0, 0)))

    qT, kvT = _qkv_call(x, Wq.T, Wk.T, Wv.T, bq[None, :], bk[None, :], bv[None, :])
    kv_flat = kvT.reshape(B * N, 2 * C)
    pp_flat = p16.reshape(B * N, 16)
    BH = 1

    idxT_h = [_topk_call(p_pad8[h * BH:(h + 1) * BH],
                         pT8[h * BH:(h + 1) * BH], h * BH) for h in range(B // BH)]
    g_h = [_gather_call(kv_flat, pp_flat, idxT_h[h].reshape(-1)) for h in range(B // BH)]
    nkv_h = [g[0].reshape(BH, K, N, 2 * C) for g in g_h]
    npp_h = [g[1].reshape(BH, K, N, 16) for g in g_h]
    p16_h = [p16[h * BH:(h + 1) * BH] for h in range(B // BH)]
    qT_h = [qT[h * BH:(h + 1) * BH] for h in range(B // BH)]

    zs = [_zstats_call(p16_h[h], npp_h[h], W1T_pad) for h in range(B // BH)]
    mean1 = sum(z[0][0] for z in zs) / BNK
    var1 = sum(z[1][0] for z in zs) / BNK - mean1 * mean1
    s1, t1 = _bn_coef(pe_g1, pe_b1, mean1, var1)

    p1 = [_p1_call(p16_h[h], npp_h[h], nkv_h[h], qT_h[h],
                   W1T_pad, s1, t1, pe_W2.T, pe_c2[None, :]) for h in range(B // BH)]
    mean2 = sum(q[2][0] for q in p1) / BNK
    var2 = sum(q[3][0] for q in p1) / BNK - mean2 * mean2
    s2, t2 = _bn_coef(at_g1, at_b1, mean2, var2)

    p2 = [_p2_call(p1[h][0], s2, t2, at_W1.T) for h in range(B // BH)]
    mu_h = sum(q[1][0] for q in p2) / BNK
    G = sum(q[2] for q in p2)
    mean3 = at_W1 @ mu_h
    e2 = jnp.sum((at_W1 @ G) * at_W1, axis=1) / BNK
    var3 = e2 - mean3 * mean3
    s3, t3 = _bn_coef(at_g2, at_b2, mean3, var3)

    y_h = [_p3_call(p2[h][0], p1[h][1], nkv_h[h], s3, t3,
                    at_W2.T, at_c2[None, :]) for h in range(B // BH)]
    y = jnp.concatenate(y_h, axis=0)
    return (p, y)


# speedup vs baseline: 12.2599x; 1.0144x over previous
import functools

import jax
import jax.numpy as jnp
from jax import lax
from jax.experimental import pallas as pl
from jax.experimental.pallas import tpu as pltpu
from jax.experimental.pallas import tpu_sc as plsc

B, N, C, K, CD = 4, 4096, 128, 16, 3
MN = 2048
NB = 512
BIG = 1e30
EPS = 1e-5
BNK = B * N * K


def _qkv_body(x_ref, wqt_ref, wkt_ref, wvt_ref, bq_ref, bk_ref, bv_ref,
              qT_ref, kvT_ref):
    xt = x_ref[0].T
    qT_ref[0] = jnp.dot(xt, wqt_ref[...], preferred_element_type=jnp.float32) + bq_ref[...]
    kvT_ref[0, :, 0:C] = jnp.dot(xt, wkt_ref[...], preferred_element_type=jnp.float32) + bk_ref[...]
    kvT_ref[0, :, C:2 * C] = jnp.dot(xt, wvt_ref[...], preferred_element_type=jnp.float32) + bv_ref[...]


def _qkv_call(x, WqT, WkT, WvT, bq, bk, bv):
    return pl.pallas_call(
        _qkv_body,
        grid=(B, N // NB),
        in_specs=[
            pl.BlockSpec((1, C, NB), lambda b, n: (b, 0, n)),
            pl.BlockSpec((C, C), lambda b, n: (0, 0)),
            pl.BlockSpec((C, C), lambda b, n: (0, 0)),
            pl.BlockSpec((C, C), lambda b, n: (0, 0)),
            pl.BlockSpec((1, C), lambda b, n: (0, 0)),
            pl.BlockSpec((1, C), lambda b, n: (0, 0)),
            pl.BlockSpec((1, C), lambda b, n: (0, 0)),
        ],
        out_specs=[
            pl.BlockSpec((1, NB, C), lambda b, n: (b, n, 0)),
            pl.BlockSpec((1, NB, 2 * C), lambda b, n: (b, n, 0)),
        ],
        out_shape=[
            jax.ShapeDtypeStruct((B, N, C), jnp.float32),
            jax.ShapeDtypeStruct((B, N, 2 * C), jnp.float32),
        ],
        compiler_params=pltpu.CompilerParams(
            dimension_semantics=("arbitrary", "arbitrary")),
    )(x, WqT, WkT, WvT, bq, bk, bv)


def _topk_body(boff, pa_ref, prt_ref, idx_ref, dist_ref):
    b = pl.program_id(0) + boff
    pa = pa_ref[0]
    prt = prt_ref[0]
    sq_all = jnp.sum(pa * pa, axis=1, keepdims=True)
    sq_rows = jnp.sum(prt * prt, axis=0, keepdims=True)
    inner = jnp.dot(pa, prt, preferred_element_type=jnp.float32)
    d = jnp.maximum(sq_all - 2.0 * inner + sq_rows, 0.0)
    zmask = jnp.all(pa == 0.0, axis=1, keepdims=True)
    dist_ref[...] = jnp.where(zmask, BIG, d)
    iota0 = lax.broadcasted_iota(jnp.int32, (N, MN), 0)
    rows16 = lax.broadcasted_iota(jnp.int32, (16, MN), 0)

    def step(kk, acc):
        dd = dist_ref[...]
        m = jnp.min(dd, axis=0, keepdims=True)
        sel = jnp.where(dd == m, iota0, N)
        j = jnp.min(sel, axis=0, keepdims=True)
        dist_ref[...] = jnp.where(iota0 == j, BIG, dd)
        return jnp.where(rows16 == kk, j + b * N, acc)

    idx_ref[0] = lax.fori_loop(0, 16, step, jnp.zeros((16, MN), jnp.int32))


def _topk_call(p_pad8, pT8, boff):
    nb = p_pad8.shape[0]
    return pl.pallas_call(
        functools.partial(_topk_body, boff),
        grid=(nb, N // MN),
        in_specs=[
            pl.BlockSpec((1, N, 8), lambda b, n: (b, 0, 0)),
            pl.BlockSpec((1, 8, MN), lambda b, n: (b, 0, n)),
        ],
        out_specs=pl.BlockSpec((1, 16, MN), lambda b, n: (b, 0, n)),
        out_shape=jax.ShapeDtypeStruct((p_pad8.shape[0], 16, N), jnp.int32),
        scratch_shapes=[pltpu.VMEM((N, MN), jnp.float32)],
        compiler_params=pltpu.CompilerParams(
            dimension_semantics=("arbitrary", "arbitrary")),
    )(p_pad8, pT8)


_NW = 32
_CH = 128


def _gather_body(rows_w, kv_hbm, pp_hbm, idx_hbm, nkv_hbm, np_hbm,
                 idxv, kvbuf, ppbuf, sem1, sem2):
    cid = lax.axis_index("c")
    sid = lax.axis_index("s")
    wid = sid * 2 + cid
    base = wid * rows_w

    def body(i, carry):
        off = base + i * _CH
        pltpu.sync_copy(idx_hbm.at[pl.ds(off, _CH)], idxv)
        cp1 = pltpu.async_copy(kv_hbm.at[idxv], kvbuf, sem1)
        cp2 = pltpu.async_copy(pp_hbm.at[idxv], ppbuf, sem2)
        cp1.wait()
        cp2.wait()
        pltpu.sync_copy(kvbuf, nkv_hbm.at[pl.ds(off, _CH)])
        pltpu.sync_copy(ppbuf, np_hbm.at[pl.ds(off, _CH)])
        return carry

    lax.fori_loop(0, rows_w // _CH, body, 0)


def _gather_call(kv_flat, pp_flat, idx_flat):
    rows = idx_flat.shape[0]
    mesh = plsc.VectorSubcoreMesh(core_axis_name="c", subcore_axis_name="s")
    fn = pl.kernel(
        functools.partial(_gather_body, rows // _NW),
        out_type=[
            jax.ShapeDtypeStruct((rows, 2 * C), jnp.float32),
            jax.ShapeDtypeStruct((rows, 16), jnp.float32),
        ],
        mesh=mesh,
        scratch_types=[
            pltpu.VMEM((_CH,), jnp.int32),
            pltpu.VMEM((_CH, 2 * C), jnp.float32),
            pltpu.VMEM((_CH, 16), jnp.float32),
            pltpu.SemaphoreType.DMA,
            pltpu.SemaphoreType.DMA,
        ],
        compiler_params=pltpu.CompilerParams(use_tc_tiling_on_sc=False),
    )
    return fn(kv_flat, pp_flat, idx_flat)


def _zstats_body(p16_ref, npp_ref, w1t_ref, sumz_ref, sumsq_ref):
    @pl.when(jnp.logical_and(pl.program_id(0) == 0, pl.program_id(1) == 0))
    def _():
        sumz_ref[...] = jnp.zeros_like(sumz_ref)
        sumsq_ref[...] = jnp.zeros_like(sumsq_ref)

    ssum = jnp.zeros((1, C), jnp.float32)
    ssq = jnp.zeros((1, C), jnp.float32)
    prow = p16_ref[0]
    for k in range(K):
        r_k = prow - npp_ref[0, k]
        z = jnp.dot(r_k, w1t_ref[...], preferred_element_type=jnp.float32)
        ssum = ssum + jnp.sum(z, axis=0, keepdims=True)
        ssq = ssq + jnp.sum(z * z, axis=0, keepdims=True)
    sumz_ref[...] += ssum
    sumsq_ref[...] += ssq


def _zstats_call(p16, npp, W1T_pad):
    return pl.pallas_call(
        _zstats_body,
        grid=(p16.shape[0], N // NB),
        in_specs=[
            pl.BlockSpec((1, NB, 16), lambda b, n: (b, n, 0)),
            pl.BlockSpec((1, K, NB, 16), lambda b, n: (b, 0, n, 0)),
            pl.BlockSpec((16, C), lambda b, n: (0, 0)),
        ],
        out_specs=[
            pl.BlockSpec((1, C), lambda b, n: (0, 0)),
            pl.BlockSpec((1, C), lambda b, n: (0, 0)),
        ],
        out_shape=[
            jax.ShapeDtypeStruct((1, C), jnp.float32),
            jax.ShapeDtypeStruct((1, C), jnp.float32),
        ],
        compiler_params=pltpu.CompilerParams(
            dimension_semantics=("arbitrary", "arbitrary")),
    )(p16, npp, W1T_pad)


def _p1_body(p16_ref, npp_ref, nkv_ref, qT_ref, w1t_ref, s1_ref, t1_ref,
             w2t_ref, c2_ref, apre_ref, nr_ref, sum2_ref, sumsq2_ref):
    @pl.when(jnp.logical_and(pl.program_id(0) == 0, pl.program_id(1) == 0))
    def _():
        sum2_ref[...] = jnp.zeros_like(sum2_ref)
        sumsq2_ref[...] = jnp.zeros_like(sumsq2_ref)

    prow = p16_ref[0]
    qrow = qT_ref[0]
    s1 = s1_ref[...]
    t1 = t1_ref[...]
    c2 = c2_ref[...]
    ssum = jnp.zeros((1, C), jnp.float32)
    ssq = jnp.zeros((1, C), jnp.float32)
    for k in range(K):
        r_k = prow - npp_ref[0, k]
        z = jnp.dot(r_k, w1t_ref[...], preferred_element_type=jnp.float32)
        h1 = jnp.maximum(z * s1 + t1, 0.0)
        nr_k = jnp.dot(h1, w2t_ref[...], preferred_element_type=jnp.float32) + c2
        apre_k = qrow - nkv_ref[0, k] + nr_k
        nr_ref[0, k] = nr_k
        apre_ref[0, k] = apre_k
        ssum = ssum + jnp.sum(apre_k, axis=0, keepdims=True)
        ssq = ssq + jnp.sum(apre_k * apre_k, axis=0, keepdims=True)
    sum2_ref[...] += ssum
    sumsq2_ref[...] += ssq


def _p1_call(p16, npp, nkv, qT, W1T_pad, s1, t1, W2T, c2):
    nb = p16.shape[0]
    return pl.pallas_call(
        _p1_body,
        grid=(nb, N // NB),
        in_specs=[
            pl.BlockSpec((1, NB, 16), lambda b, n: (b, n, 0)),
            pl.BlockSpec((1, K, NB, 16), lambda b, n: (b, 0, n, 0)),
            pl.BlockSpec((1, K, NB, C), lambda b, n: (b, 0, n, 0)),
            pl.BlockSpec((1, NB, C), lambda b, n: (b, n, 0)),
            pl.BlockSpec((16, C), lambda b, n: (0, 0)),
            pl.BlockSpec((1, C), lambda b, n: (0, 0)),
            pl.BlockSpec((1, C), lambda b, n: (0, 0)),
            pl.BlockSpec((C, C), lambda b, n: (0, 0)),
            pl.BlockSpec((1, C), lambda b, n: (0, 0)),
        ],
        out_specs=[
            pl.BlockSpec((1, K, NB, C), lambda b, n: (b, 0, n, 0)),
            pl.BlockSpec((1, K, NB, C), lambda b, n: (b, 0, n, 0)),
            pl.BlockSpec((1, C), lambda b, n: (0, 0)),
            pl.BlockSpec((1, C), lambda b, n: (0, 0)),
        ],
        out_shape=[
            jax.ShapeDtypeStruct((nb, K, N, C), jnp.float32),
            jax.ShapeDtypeStruct((nb, K, N, C), jnp.float32),
            jax.ShapeDtypeStruct((1, C), jnp.float32),
            jax.ShapeDtypeStruct((1, C), jnp.float32),
        ],
        compiler_params=pltpu.CompilerParams(
            dimension_semantics=("arbitrary", "arbitrary")),
    )(p16, npp, nkv, qT, W1T_pad, s1, t1, W2T, c2)


def _p2_body(apre_ref, s2_ref, t2_ref, w1t_ref, a1_ref, sumh_ref, g_ref):
    @pl.when(jnp.logical_and(pl.program_id(0) == 0, pl.program_id(1) == 0))
    def _():
        sumh_ref[...] = jnp.zeros_like(sumh_ref)
        g_ref[...] = jnp.zeros_like(g_ref)

    s2 = s2_ref[...]
    t2 = t2_ref[...]
    ssum = jnp.zeros((1, C), jnp.float32)
    gacc = jnp.zeros((C, C), jnp.float32)
    for k in range(K):
        h = jnp.maximum(apre_ref[0, k] * s2 + t2, 0.0)
        a1_ref[0, k] = jnp.dot(h, w1t_ref[...], preferred_element_type=jnp.float32)
        ssum = ssum + jnp.sum(h, axis=0, keepdims=True)
        gacc = gacc + jnp.dot(h.T, h, preferred_element_type=jnp.float32)
    sumh_ref[...] += ssum
    g_ref[...] += gacc


def _p2_call(apre, s2, t2, atW1T):
    nb = apre.shape[0]
    return pl.pallas_call(
        _p2_body,
        grid=(nb, N // NB),
        in_specs=[
            pl.BlockSpec((1, K, NB, C), lambda b, n: (b, 0, n, 0)),
            pl.BlockSpec((1, C), lambda b, n: (0, 0)),
            pl.BlockSpec((1, C), lambda b, n: (0, 0)),
            pl.BlockSpec((C, C), lambda b, n: (0, 0)),
        ],
        out_specs=[
            pl.BlockSpec((1, K, NB, C), lambda b, n: (b, 0, n, 0)),
            pl.BlockSpec((1, C), lambda b, n: (0, 0)),
            pl.BlockSpec((C, C), lambda b, n: (0, 0)),
        ],
        out_shape=[
            jax.ShapeDtypeStruct((nb, K, N, C), jnp.float32),
            jax.ShapeDtypeStruct((1, C), jnp.float32),
            jax.ShapeDtypeStruct((C, C), jnp.float32),
        ],
        compiler_params=pltpu.CompilerParams(
            dimension_semantics=("arbitrary", "arbitrary")),
    )(apre, s2, t2, atW1T)


def _p3_body(a1_ref, nr_ref, nkv_ref, s3_ref, t3_ref, w2t_ref, c2_ref, y_ref):
    s3 = s3_ref[...]
    t3 = t3_ref[...]
    c2 = c2_ref[...]
    a2 = []
    for k in range(K):
        h = jnp.maximum(a1_ref[0, k] * s3 + t3, 0.0)
        a2.append(jnp.dot(h, w2t_ref[...], preferred_element_type=jnp.float32) + c2)
    mx = a2[0]
    for k in range(1, K):
        mx = jnp.maximum(mx, a2[k])
    esum = jnp.zeros_like(mx)
    ynum = jnp.zeros_like(mx)
    for k in range(K):
        e = jnp.exp(a2[k] - mx)
        esum = esum + e
        ynum = ynum + e * (nkv_ref[0, k] + nr_ref[0, k])
    y = ynum / esum
    y_ref[0] = y.T


def _p3_call(a1, nr, nkv, s3, t3, atW2T, atc2):
    nb = a1.shape[0]
    return pl.pallas_call(
        _p3_body,
        grid=(nb, N // NB),
        in_specs=[
            pl.BlockSpec((1, K, NB, C), lambda b, n: (b, 0, n, 0)),
            pl.BlockSpec((1, K, NB, C), lambda b, n: (b, 0, n, 0)),
            pl.BlockSpec((1, K, NB, C), lambda b, n: (b, 0, n, 1)),
            pl.BlockSpec((1, C), lambda b, n: (0, 0)),
            pl.BlockSpec((1, C), lambda b, n: (0, 0)),
            pl.BlockSpec((C, C), lambda b, n: (0, 0)),
            pl.BlockSpec((1, C), lambda b, n: (0, 0)),
        ],
        out_specs=pl.BlockSpec((1, C, NB), lambda b, n: (b, 0, n)),
        out_shape=jax.ShapeDtypeStruct((nb, C, N), jnp.float32),
        compiler_params=pltpu.CompilerParams(
            dimension_semantics=("arbitrary", "arbitrary")),
    )(a1, nr, nkv, s3, t3, atW2T, atc2)


def _bn_coef(g, b, mean, var):
    s = g / jnp.sqrt(var + EPS)
    return (s[None, :], (b - mean * s)[None, :])


def kernel(p, x, Wq, bq, Wk, bk, Wv, bv, pe_W1, pe_g1, pe_b1, pe_W2, pe_c2,
           at_g1, at_b1, at_W1, at_g2, at_b2, at_W2, at_c2):
    f32 = jnp.float32
    p = p.astype(f32)
    p_pad8 = jnp.pad(p, ((0, 0), (0, 0), (0, 8 - CD)))
    pT8 = jnp.transpose(p_pad8, (0, 2, 1))
    p16 = jnp.pad(p, ((0, 0), (0, 0), (0, 16 - CD)))
    W1T_pad = jnp.pad(pe_W1.T, ((0, 16 - CD), (0, 0)))

    qT, kvT = _qkv_call(x, Wq.T, Wk.T, Wv.T, bq[None, :], bk[None, :], bv[None, :])
    kv_flat = kvT.reshape(B * N, 2 * C)
    pp_flat = p16.reshape(B * N, 16)
    BH = 1

    idxT_h = [_topk_call(p_pad8[h * BH:(h + 1) * BH],
                         pT8[h * BH:(h + 1) * BH], h * BH) for h in range(B // BH)]
    g_h = [_gather_call(kv_flat, pp_flat, idxT_h[h].reshape(-1)) for h in range(B // BH)]
    nkv_h = [g[0].reshape(BH, K, N, 2 * C) for g in g_h]
    npp_h = [g[1].reshape(BH, K, N, 16) for g in g_h]
    p16_h = [p16[h * BH:(h + 1) * BH] for h in range(B // BH)]
    qT_h = [qT[h * BH:(h + 1) * BH] for h in range(B // BH)]

    zs = [_zstats_call(p16_h[h], npp_h[h], W1T_pad) for h in range(B // BH)]
    mean1 = sum(z[0][0] for z in zs) / BNK
    var1 = sum(z[1][0] for z in zs) / BNK - mean1 * mean1
    s1, t1 = _bn_coef(pe_g1, pe_b1, mean1, var1)

    p1 = [_p1_call(p16_h[h], npp_h[h], nkv_h[h], qT_h[h],
                   W1T_pad, s1, t1, pe_W2.T, pe_c2[None, :]) for h in range(B // BH)]
    mean2 = sum(q[2][0] for q in p1) / BNK
    var2 = sum(q[3][0] for q in p1) / BNK - mean2 * mean2
    s2, t2 = _bn_coef(at_g1, at_b1, mean2, var2)

    p2 = [_p2_call(p1[h][0], s2, t2, at_W1.T) for h in range(B // BH)]
    mu_h = sum(q[1][0] for q in p2) / BNK
    G = sum(q[2] for q in p2)
    mean3 = at_W1 @ mu_h
    e2 = jnp.sum((at_W1 @ G) * at_W1, axis=1) / BNK
    var3 = e2 - mean3 * mean3
    s3, t3 = _bn_coef(at_g2, at_b2, mean3, var3)

    y_h = [_p3_call(p2[h][0], p1[h][1], nkv_h[h], s3, t3,
                    at_W2.T, at_c2[None, :]) for h in range(B // BH)]
    y = jnp.concatenate(y_h, axis=0)
    return (p, y)


# speedup vs baseline: 12.2667x; 1.0006x over previous
import functools

import jax
import jax.numpy as jnp
from jax import lax
from jax.experimental import pallas as pl
from jax.experimental.pallas import tpu as pltpu
from jax.experimental.pallas import tpu_sc as plsc

B, N, C, K, CD = 4, 4096, 128, 16, 3
MN = 2048
NB = 512
BIG = 1e30
EPS = 1e-5
BNK = B * N * K


def _qkv_body(x_ref, wqt_ref, wkt_ref, wvt_ref, bq_ref, bk_ref, bv_ref,
              qT_ref, kvT_ref):
    xt = x_ref[0].T
    qT_ref[0] = jnp.dot(xt, wqt_ref[...], preferred_element_type=jnp.float32) + bq_ref[...]
    kvT_ref[0, :, 0:C] = jnp.dot(xt, wkt_ref[...], preferred_element_type=jnp.float32) + bk_ref[...]
    kvT_ref[0, :, C:2 * C] = jnp.dot(xt, wvt_ref[...], preferred_element_type=jnp.float32) + bv_ref[...]


def _qkv_call(x, WqT, WkT, WvT, bq, bk, bv):
    return pl.pallas_call(
        _qkv_body,
        grid=(B, N // NB),
        in_specs=[
            pl.BlockSpec((1, C, NB), lambda b, n: (b, 0, n)),
            pl.BlockSpec((C, C), lambda b, n: (0, 0)),
            pl.BlockSpec((C, C), lambda b, n: (0, 0)),
            pl.BlockSpec((C, C), lambda b, n: (0, 0)),
            pl.BlockSpec((1, C), lambda b, n: (0, 0)),
            pl.BlockSpec((1, C), lambda b, n: (0, 0)),
            pl.BlockSpec((1, C), lambda b, n: (0, 0)),
        ],
        out_specs=[
            pl.BlockSpec((1, NB, C), lambda b, n: (b, n, 0)),
            pl.BlockSpec((1, NB, 2 * C), lambda b, n: (b, n, 0)),
        ],
        out_shape=[
            jax.ShapeDtypeStruct((B, N, C), jnp.float32),
            jax.ShapeDtypeStruct((B, N, 2 * C), jnp.float32),
        ],
        compiler_params=pltpu.CompilerParams(
            dimension_semantics=("arbitrary", "arbitrary")),
    )(x, WqT, WkT, WvT, bq, bk, bv)


def _topk_body(boff, pa_ref, prt_ref, idx_ref, dist_ref):
    b = pl.program_id(0) + boff
    pa = pa_ref[0]
    prt = prt_ref[0]
    sq_all = jnp.sum(pa * pa, axis=1, keepdims=True)
    sq_rows = jnp.sum(prt * prt, axis=0, keepdims=True)
    inner = jnp.dot(pa, prt, preferred_element_type=jnp.float32)
    d = jnp.maximum(sq_all - 2.0 * inner + sq_rows, 0.0)
    zmask = jnp.all(pa == 0.0, axis=1, keepdims=True)
    dist_ref[...] = jnp.where(zmask, BIG, d)
    iota0 = lax.broadcasted_iota(jnp.int32, (N, MN), 0)
    rows16 = lax.broadcasted_iota(jnp.int32, (16, MN), 0)

    def step(kk, acc):
        dd = dist_ref[...]
        m = jnp.min(dd, axis=0, keepdims=True)
        sel = jnp.where(dd == m, iota0, N)
        j = jnp.min(sel, axis=0, keepdims=True)
        dist_ref[...] = jnp.where(iota0 == j, BIG, dd)
        return jnp.where(rows16 == kk, j + b * N, acc)

    idx_ref[0] = lax.fori_loop(0, 16, step, jnp.zeros((16, MN), jnp.int32))


def _topk_call(p_pad8, pT8, boff):
    nb = p_pad8.shape[0]
    return pl.pallas_call(
        functools.partial(_topk_body, boff),
        grid=(nb, N // MN),
        in_specs=[
            pl.BlockSpec((1, N, 8), lambda b, n: (b, 0, 0)),
            pl.BlockSpec((1, 8, MN), lambda b, n: (b, 0, n)),
        ],
        out_specs=pl.BlockSpec((1, 16, MN), lambda b, n: (b, 0, n)),
        out_shape=jax.ShapeDtypeStruct((p_pad8.shape[0], 16, N), jnp.int32),
        scratch_shapes=[pltpu.VMEM((N, MN), jnp.float32)],
        compiler_params=pltpu.CompilerParams(
            dimension_semantics=("arbitrary", "arbitrary")),
    )(p_pad8, pT8)


_NW = 32
_CH = 256


def _gather_body(rows_w, kv_hbm, pp_hbm, idx_hbm, nkv_hbm, np_hbm,
                 idxv, kvbuf, ppbuf, sem1, sem2):
    cid = lax.axis_index("c")
    sid = lax.axis_index("s")
    wid = sid * 2 + cid
    base = wid * rows_w

    def body(i, carry):
        off = base + i * _CH
        pltpu.sync_copy(idx_hbm.at[pl.ds(off, _CH)], idxv)
        cp1 = pltpu.async_copy(kv_hbm.at[idxv], kvbuf, sem1)
        cp2 = pltpu.async_copy(pp_hbm.at[idxv], ppbuf, sem2)
        cp1.wait()
        cp2.wait()
        pltpu.sync_copy(kvbuf, nkv_hbm.at[pl.ds(off, _CH)])
        pltpu.sync_copy(ppbuf, np_hbm.at[pl.ds(off, _CH)])
        return carry

    lax.fori_loop(0, rows_w // _CH, body, 0)


def _gather_call(kv_flat, pp_flat, idx_flat):
    rows = idx_flat.shape[0]
    mesh = plsc.VectorSubcoreMesh(core_axis_name="c", subcore_axis_name="s")
    fn = pl.kernel(
        functools.partial(_gather_body, rows // _NW),
        out_type=[
            jax.ShapeDtypeStruct((rows, 2 * C), jnp.float32),
            jax.ShapeDtypeStruct((rows, 16), jnp.float32),
        ],
        mesh=mesh,
        scratch_types=[
            pltpu.VMEM((_CH,), jnp.int32),
            pltpu.VMEM((_CH, 2 * C), jnp.float32),
            pltpu.VMEM((_CH, 16), jnp.float32),
            pltpu.SemaphoreType.DMA,
            pltpu.SemaphoreType.DMA,
        ],
        compiler_params=pltpu.CompilerParams(use_tc_tiling_on_sc=False),
    )
    return fn(kv_flat, pp_flat, idx_flat)


def _zstats_body(p16_ref, npp_ref, w1t_ref, sumz_ref, sumsq_ref):
    @pl.when(jnp.logical_and(pl.program_id(0) == 0, pl.program_id(1) == 0))
    def _():
        sumz_ref[...] = jnp.zeros_like(sumz_ref)
        sumsq_ref[...] = jnp.zeros_like(sumsq_ref)

    ssum = jnp.zeros((1, C), jnp.float32)
    ssq = jnp.zeros((1, C), jnp.float32)
    prow = p16_ref[0]
    for k in range(K):
        r_k = prow - npp_ref[0, k]
        z = jnp.dot(r_k, w1t_ref[...], preferred_element_type=jnp.float32)
        ssum = ssum + jnp.sum(z, axis=0, keepdims=True)
        ssq = ssq + jnp.sum(z * z, axis=0, keepdims=True)
    sumz_ref[...] += ssum
    sumsq_ref[...] += ssq


def _zstats_call(p16, npp, W1T_pad):
    return pl.pallas_call(
        _zstats_body,
        grid=(p16.shape[0], N // NB),
        in_specs=[
            pl.BlockSpec((1, NB, 16), lambda b, n: (b, n, 0)),
            pl.BlockSpec((1, K, NB, 16), lambda b, n: (b, 0, n, 0)),
            pl.BlockSpec((16, C), lambda b, n: (0, 0)),
        ],
        out_specs=[
            pl.BlockSpec((1, C), lambda b, n: (0, 0)),
            pl.BlockSpec((1, C), lambda b, n: (0, 0)),
        ],
        out_shape=[
            jax.ShapeDtypeStruct((1, C), jnp.float32),
            jax.ShapeDtypeStruct((1, C), jnp.float32),
        ],
        compiler_params=pltpu.CompilerParams(
            dimension_semantics=("arbitrary", "arbitrary")),
    )(p16, npp, W1T_pad)


def _p1_body(p16_ref, npp_ref, nkv_ref, qT_ref, w1t_ref, s1_ref, t1_ref,
             w2t_ref, c2_ref, apre_ref, nr_ref, sum2_ref, sumsq2_ref):
    @pl.when(jnp.logical_and(pl.program_id(0) == 0, pl.program_id(1) == 0))
    def _():
        sum2_ref[...] = jnp.zeros_like(sum2_ref)
        sumsq2_ref[...] = jnp.zeros_like(sumsq2_ref)

    prow = p16_ref[0]
    qrow = qT_ref[0]
    s1 = s1_ref[...]
    t1 = t1_ref[...]
    c2 = c2_ref[...]
    ssum = jnp.zeros((1, C), jnp.float32)
    ssq = jnp.zeros((1, C), jnp.float32)
    for k in range(K):
        r_k = prow - npp_ref[0, k]
        z = jnp.dot(r_k, w1t_ref[...], preferred_element_type=jnp.float32)
        h1 = jnp.maximum(z * s1 + t1, 0.0)
        nr_k = jnp.dot(h1, w2t_ref[...], preferred_element_type=jnp.float32) + c2
        apre_k = qrow - nkv_ref[0, k] + nr_k
        nr_ref[0, k] = nr_k
        apre_ref[0, k] = apre_k
        ssum = ssum + jnp.sum(apre_k, axis=0, keepdims=True)
        ssq = ssq + jnp.sum(apre_k * apre_k, axis=0, keepdims=True)
    sum2_ref[...] += ssum
    sumsq2_ref[...] += ssq


def _p1_call(p16, npp, nkv, qT, W1T_pad, s1, t1, W2T, c2):
    nb = p16.shape[0]
    return pl.pallas_call(
        _p1_body,
        grid=(nb, N // NB),
        in_specs=[
            pl.BlockSpec((1, NB, 16), lambda b, n: (b, n, 0)),
            pl.BlockSpec((1, K, NB, 16), lambda b, n: (b, 0, n, 0)),
            pl.BlockSpec((1, K, NB, C), lambda b, n: (b, 0, n, 0)),
            pl.BlockSpec((1, NB, C), lambda b, n: (b, n, 0)),
            pl.BlockSpec((16, C), lambda b, n: (0, 0)),
            pl.BlockSpec((1, C), lambda b, n: (0, 0)),
            pl.BlockSpec((1, C), lambda b, n: (0, 0)),
            pl.BlockSpec((C, C), lambda b, n: (0, 0)),
            pl.BlockSpec((1, C), lambda b, n: (0, 0)),
        ],
        out_specs=[
            pl.BlockSpec((1, K, NB, C), lambda b, n: (b, 0, n, 0)),
            pl.BlockSpec((1, K, NB, C), lambda b, n: (b, 0, n, 0)),
            pl.BlockSpec((1, C), lambda b, n: (0, 0)),
            pl.BlockSpec((1, C), lambda b, n: (0, 0)),
        ],
        out_shape=[
            jax.ShapeDtypeStruct((nb, K, N, C), jnp.float32),
            jax.ShapeDtypeStruct((nb, K, N, C), jnp.float32),
            jax.ShapeDtypeStruct((1, C), jnp.float32),
            jax.ShapeDtypeStruct((1, C), jnp.float32),
        ],
        compiler_params=pltpu.CompilerParams(
            dimension_semantics=("arbitrary", "arbitrary")),
    )(p16, npp, nkv, qT, W1T_pad, s1, t1, W2T, c2)


def _p2_body(apre_ref, s2_ref, t2_ref, w1t_ref, a1_ref, sumh_ref, g_ref):
    @pl.when(jnp.logical_and(pl.program_id(0) == 0, pl.program_id(1) == 0))
    def _():
        sumh_ref[...] = jnp.zeros_like(sumh_ref)
        g_ref[...] = jnp.zeros_like(g_ref)

    s2 = s2_ref[...]
    t2 = t2_ref[...]
    ssum = jnp.zeros((1, C), jnp.float32)
    gacc = jnp.zeros((C, C), jnp.float32)
    for k in range(K):
        h = jnp.maximum(apre_ref[0, k] * s2 + t2, 0.0)
        a1_ref[0, k] = jnp.dot(h, w1t_ref[...], preferred_element_type=jnp.float32)
        ssum = ssum + jnp.sum(h, axis=0, keepdims=True)
        gacc = gacc + jnp.dot(h.T, h, preferred_element_type=jnp.float32)
    sumh_ref[...] += ssum
    g_ref[...] += gacc


def _p2_call(apre, s2, t2, atW1T):
    nb = apre.shape[0]
    return pl.pallas_call(
        _p2_body,
        grid=(nb, N // NB),
        in_specs=[
            pl.BlockSpec((1, K, NB, C), lambda b, n: (b, 0, n, 0)),
            pl.BlockSpec((1, C), lambda b, n: (0, 0)),
            pl.BlockSpec((1, C), lambda b, n: (0, 0)),
            pl.BlockSpec((C, C), lambda b, n: (0, 0)),
        ],
        out_specs=[
            pl.BlockSpec((1, K, NB, C), lambda b, n: (b, 0, n, 0)),
            pl.BlockSpec((1, C), lambda b, n: (0, 0)),
            pl.BlockSpec((C, C), lambda b, n: (0, 0)),
        ],
        out_shape=[
            jax.ShapeDtypeStruct((nb, K, N, C), jnp.float32),
            jax.ShapeDtypeStruct((1, C), jnp.float32),
            jax.ShapeDtypeStruct((C, C), jnp.float32),
        ],
        compiler_params=pltpu.CompilerParams(
            dimension_semantics=("arbitrary", "arbitrary")),
    )(apre, s2, t2, atW1T)


def _p3_body(a1_ref, nr_ref, nkv_ref, s3_ref, t3_ref, w2t_ref, c2_ref, y_ref):
    s3 = s3_ref[...]
    t3 = t3_ref[...]
    c2 = c2_ref[...]
    a2 = []
    for k in range(K):
        h = jnp.maximum(a1_ref[0, k] * s3 + t3, 0.0)
        a2.append(jnp.dot(h, w2t_ref[...], preferred_element_type=jnp.float32) + c2)
    mx = a2[0]
    for k in range(1, K):
        mx = jnp.maximum(mx, a2[k])
    esum = jnp.zeros_like(mx)
    ynum = jnp.zeros_like(mx)
    for k in range(K):
        e = jnp.exp(a2[k] - mx)
        esum = esum + e
        ynum = ynum + e * (nkv_ref[0, k] + nr_ref[0, k])
    y = ynum / esum
    y_ref[0] = y.T


def _p3_call(a1, nr, nkv, s3, t3, atW2T, atc2):
    nb = a1.shape[0]
    return pl.pallas_call(
        _p3_body,
        grid=(nb, N // NB),
        in_specs=[
            pl.BlockSpec((1, K, NB, C), lambda b, n: (b, 0, n, 0)),
            pl.BlockSpec((1, K, NB, C), lambda b, n: (b, 0, n, 0)),
            pl.BlockSpec((1, K, NB, C), lambda b, n: (b, 0, n, 1)),
            pl.BlockSpec((1, C), lambda b, n: (0, 0)),
            pl.BlockSpec((1, C), lambda b, n: (0, 0)),
            pl.BlockSpec((C, C), lambda b, n: (0, 0)),
            pl.BlockSpec((1, C), lambda b, n: (0, 0)),
        ],
        out_specs=pl.BlockSpec((1, C, NB), lambda b, n: (b, 0, n)),
        out_shape=jax.ShapeDtypeStruct((nb, C, N), jnp.float32),
        compiler_params=pltpu.CompilerParams(
            dimension_semantics=("arbitrary", "arbitrary")),
    )(a1, nr, nkv, s3, t3, atW2T, atc2)


def _bn_coef(g, b, mean, var):
    s = g / jnp.sqrt(var + EPS)
    return (s[None, :], (b - mean * s)[None, :])


def kernel(p, x, Wq, bq, Wk, bk, Wv, bv, pe_W1, pe_g1, pe_b1, pe_W2, pe_c2,
           at_g1, at_b1, at_W1, at_g2, at_b2, at_W2, at_c2):
    f32 = jnp.float32
    p = p.astype(f32)
    p_pad8 = jnp.pad(p, ((0, 0), (0, 0), (0, 8 - CD)))
    pT8 = jnp.transpose(p_pad8, (0, 2, 1))
    p16 = jnp.pad(p, ((0, 0), (0, 0), (0, 16 - CD)))
    W1T_pad = jnp.pad(pe_W1.T, ((0, 16 - CD), (0, 0)))

    qT, kvT = _qkv_call(x, Wq.T, Wk.T, Wv.T, bq[None, :], bk[None, :], bv[None, :])
    kv_flat = kvT.reshape(B * N, 2 * C)
    pp_flat = p16.reshape(B * N, 16)
    BH = 1

    idxT_h = [_topk_call(p_pad8[h * BH:(h + 1) * BH],
                         pT8[h * BH:(h + 1) * BH], h * BH) for h in range(B // BH)]
    g_h = [_gather_call(kv_flat, pp_flat, idxT_h[h].reshape(-1)) for h in range(B // BH)]
    nkv_h = [g[0].reshape(BH, K, N, 2 * C) for g in g_h]
    npp_h = [g[1].reshape(BH, K, N, 16) for g in g_h]
    p16_h = [p16[h * BH:(h + 1) * BH] for h in range(B // BH)]
    qT_h = [qT[h * BH:(h + 1) * BH] for h in range(B // BH)]

    zs = [_zstats_call(p16_h[h], npp_h[h], W1T_pad) for h in range(B // BH)]
    mean1 = sum(z[0][0] for z in zs) / BNK
    var1 = sum(z[1][0] for z in zs) / BNK - mean1 * mean1
    s1, t1 = _bn_coef(pe_g1, pe_b1, mean1, var1)

    p1 = [_p1_call(p16_h[h], npp_h[h], nkv_h[h], qT_h[h],
                   W1T_pad, s1, t1, pe_W2.T, pe_c2[None, :]) for h in range(B // BH)]
    mean2 = sum(q[2][0] for q in p1) / BNK
    var2 = sum(q[3][0] for q in p1) / BNK - mean2 * mean2
    s2, t2 = _bn_coef(at_g1, at_b1, mean2, var2)

    p2 = [_p2_call(p1[h][0], s2, t2, at_W1.T) for h in range(B // BH)]
    mu_h = sum(q[1][0] for q in p2) / BNK
    G = sum(q[2] for q in p2)
    mean3 = at_W1 @ mu_h
    e2 = jnp.sum((at_W1 @ G) * at_W1, axis=1) / BNK
    var3 = e2 - mean3 * mean3
    s3, t3 = _bn_coef(at_g2, at_b2, mean3, var3)

    y_h = [_p3_call(p2[h][0], p1[h][1], nkv_h[h], s3, t3,
                    at_W2.T, at_c2[None, :]) for h in range(B // BH)]
    y = jnp.concatenate(y_h, axis=0)
    return (p, y)
